# Initial kernel scaffold; baseline (speedup 1.0000x reference)
import jax
import jax.numpy as jnp
from jax.experimental import pallas as pl


def kernel(target_nodes, metapath_list, node_type_mapping, node_feature_list, W_proj, b_proj, attn, W_sem, b_sem, a_sem, W_cls, b_cls):
    raise NotImplementedError("write your pallas kernel here")


# speedup vs baseline: 46.4090x; 46.4090x over previous
import jax
import jax.numpy as jnp
from jax import lax
from jax.experimental import pallas as pl
from jax.experimental.pallas import tpu as pltpu
from jax.experimental.pallas import tpu_sc as plsc

N = 10000
NPAD = 10240
DRAW = 128
D = 64
H = 4
M = 2
B = 4096
E = 160000
SLOPE = 0.01
STAGE = 1280
NSTG = E // STAGE
FB = 48
PCAP = STAGE + 2 * FB
SPT = B // 16
STW = 48
ACCW = 272
RB = 1024


def _lrelu(x):
    return jnp.where(x > 0, x, SLOPE * x)


def _pre_body(x_ref, wt_ref, aw_ref, bp_ref, mk_ref, st_ref, cs_ref):
    i = pl.program_id(0)
    nfb = (jnp.dot(x_ref[...], wt_ref[...], preferred_element_type=jnp.float32)
           + bp_ref[...]) * mk_ref[...]
    sc = jnp.dot(nfb, aw_ref[...], preferred_element_type=jnp.float32)
    st_ref[...] = jnp.concatenate(
        [nfb, sc[:, 8:16], sc[:, 0:8],
         jnp.zeros((RB, 128 - D - 16), jnp.float32)], axis=1)
    bm = jnp.max(sc, axis=0)

    @pl.when(i == 0)
    def _init():
        cs_ref[...] = jnp.full((1, 128), -1e30, jnp.float32)

    cs_ref[0:1, 0:16] = jnp.maximum(cs_ref[0:1, 0:16], bm[None, :])

    @pl.when(i == pl.num_programs(0) - 1)
    def _fin():
        v = cs_ref[0:1, 0:16]
        cs_ref[0:1, 16:24] = _lrelu(v[:, 0:8] + v[:, 8:16])


def _pre_call(xp, wt, aw, bp, maskf):
    ng = NPAD // RB
    return pl.pallas_call(
        _pre_body,
        grid=(ng,),
        in_specs=[
            pl.BlockSpec((RB, DRAW), lambda i: (i, 0)),
            pl.BlockSpec((DRAW, D), lambda i: (0, 0)),
            pl.BlockSpec((D, 16), lambda i: (0, 0)),
            pl.BlockSpec((1, D), lambda i: (0, 0)),
            pl.BlockSpec((RB, 1), lambda i: (i, 0)),
        ],
        out_specs=[
            pl.BlockSpec((RB, 128), lambda i: (i, 0)),
            pl.BlockSpec((1, 128), lambda i: (0, 0)),
        ],
        out_shape=[
            jax.ShapeDtypeStruct((NPAD, 128), jnp.float32),
            jax.ShapeDtypeStruct((1, 128), jnp.float32),
        ],
    )(xp, wt, aw, bp, maskf)


def _edge_body(tgt_hbm, nb0_hbm, cur0_hbm, nb1_hbm, cur1_hbm, st_hbm, cs_hbm,
               hmn0_hbm, hmn1_hbm, mrow_hbm,
               map_v, acc_v, pnb_v, pcur_v, rs_v, rd_v,
               nb_v, cur_v, tch_v, mch_v, cs_v, drn_v, sem):
    cid = lax.axis_index("c")
    sid = lax.axis_index("s")
    wid = cid * 16 + sid
    lanes = lax.iota(jnp.int32, 16)
    zpad = jnp.zeros((16,), jnp.float32)
    sent16 = jnp.full((16,), NPAD - 1, jnp.int32)

    pltpu.sync_copy(cs_hbm.at[0, pl.ds(0, 32)], cs_v)

    minus1 = jnp.full((16,), -1, jnp.int32)

    def _init_map(i, c):
        map_v[pl.ds(pl.multiple_of(i * 16, 16), 16)] = minus1
        return c

    lax.fori_loop(0, NPAD // 16, _init_map, 0)

    def _scat_map(j, c):
        pltpu.sync_copy(tgt_hbm.at[pl.ds(pl.multiple_of(j * 64, 64), 64)],
                        tch_v)
        for g in range(4):
            t16 = tch_v[pl.ds(g * 16, 16)]
            plsc.store_scatter(map_v, [t16], lanes + j * 64 + g * 16)
        return c

    lax.fori_loop(0, B // 64, _scat_map, 0)

    @pl.when(wid == 0)
    def _mrow():
        def body(j, c):
            jb = pl.multiple_of(j * 64, 64)
            pltpu.sync_copy(tgt_hbm.at[pl.ds(jb, 64)], tch_v)
            for g in range(4):
                t16 = tch_v[pl.ds(g * 16, 16)]
                mch_v[pl.ds(g * 16, 16)] = plsc.load_gather(map_v, [t16])
            pltpu.sync_copy(mch_v, mrow_hbm.at[pl.ds(jb, 64)])
            return c

        lax.fori_loop(0, B // 64, body, 0)

    def _zero_acc(i, c):
        acc_v[pl.ds(i * 16, 16)] = zpad
        return c

    lax.fori_loop(0, SPT * ACCW // 16, _zero_acc, 0)

    nb_hbm = (nb0_hbm, nb1_hbm)
    cur_hbm = (cur0_hbm, cur1_hbm)
    sid16 = jnp.zeros((16,), jnp.int32) + sid

    def _flush(b, cc):
        fb = pl.multiple_of(b * FB, FB)
        pltpu.async_copy(st_hbm.at[pnb_v.at[pl.ds(fb, FB)]], rs_v, sem).wait()
        pltpu.async_copy(st_hbm.at[pcur_v.at[pl.ds(fb, FB)]], rd_v,
                         sem).wait()
        for g in range(FB // 16):
            go = g * 16
            cur16 = pcur_v[pl.ds(fb + go, 16)]
            m16 = plsc.load_gather(map_v, [cur16])
            own = jnp.right_shift(m16, 8) == sid16
            slot16 = jnp.bitwise_and(m16, 255)
            kvec = lanes + go
            ees = []
            for h in range(H):
                col_ss = jnp.zeros((16,), jnp.int32) + (cid * 4 + D + h)
                col_sd = jnp.zeros((16,), jnp.int32) + (cid * 4 + D + 8 + h)
                ss = plsc.load_gather(rs_v, [kvec, col_ss])
                sd = plsc.load_gather(rd_v, [kvec, col_sd])
                csp = plsc.load_gather(
                    cs_v, [jnp.zeros((16,), jnp.int32) + (cid * 4 + 16 + h)])
                ee = jnp.exp(_lrelu(sd + ss) - csp)
                ees.append(jnp.where(own, ee, 0.0))
            for l in range(16):
                k = go + l
                sb2 = slot16[l] * ACCW
                feats = [rs_v[k, pl.ds(j * 16, 16)] for j in range(4)]
                for h in range(H):
                    eh = ees[h][l]
                    for j in range(4):
                        colsl = pl.ds(sb2 + h * D + j * 16, 16)
                        acc_v[colsl] = acc_v[colsl] + eh * feats[j]
                dsl = pl.ds(sb2 + 256, 16)
                evl = [jnp.where(lanes == 0, ees[0][l], 0.0)]
                for h in range(1, H):
                    evl.append(jnp.where(lanes == h, ees[h][l],
                                         evl[h - 1]))
                acc_v[dsl] = acc_v[dsl] + evl[H - 1]
        return cc

    def _stage(st, cnt):
        sb = pl.multiple_of(st * STAGE, STAGE)
        for c in range(2):
            @pl.when(cid == c)
            def _cp(c=c):
                pltpu.sync_copy(nb_hbm[c].at[pl.ds(sb, STAGE)], nb_v)
                pltpu.sync_copy(cur_hbm[c].at[pl.ds(sb, STAGE)], cur_v)

        def _scan(g, cn):
            go = pl.multiple_of(g * 16, 16)
            cur16 = cur_v[pl.ds(go, 16)]
            m16 = plsc.load_gather(map_v, [cur16])
            own = jnp.right_shift(m16, 8) == sid16
            csum = plsc.cumsum(jnp.where(own, jnp.int32(1), jnp.int32(0)))
            pos16 = cn + csum - 1
            plsc.store_scatter(pnb_v, [pos16], nb_v[pl.ds(go, 16)], mask=own)
            plsc.store_scatter(pcur_v, [pos16], cur16, mask=own)
            return cn + csum[15]

        cnt = lax.fori_loop(0, STAGE // 16, _scan, cnt)
        nfull = cnt // FB
        lax.fori_loop(0, nfull, _flush, 0)
        rb = pl.multiple_of(nfull * FB, FB)
        for g in range(FB // 16):
            tnb = pnb_v[pl.ds(rb + g * 16, 16)]
            tcur = pcur_v[pl.ds(rb + g * 16, 16)]
            pnb_v[pl.ds(g * 16, 16)] = tnb
            pcur_v[pl.ds(g * 16, 16)] = tcur
        return cnt - nfull * FB

    cnt = lax.fori_loop(0, NSTG, _stage, jnp.int32(0))
    for p in range(FB // 16):
        plsc.store_scatter(pnb_v, [cnt + p * 16 + lanes], sent16)
        plsc.store_scatter(pcur_v, [cnt + p * 16 + lanes], sent16)
    lax.fori_loop(0, (cnt + FB - 1) // FB, _flush, 0)

    def _norm(g, c):
        def _row(l, cc):
            kb = (g * 2 + l) * ACCW
            dvec = acc_v[pl.ds(kb + 256, 16)]
            ivec = jnp.where(dvec > 0, 1.0 / dvec, 0.0)
            for h in range(H):
                inv = ivec[h]
                for j in range(4):
                    f = h * D + j * 16
                    drn_v[l, pl.ds(f, 16)] = acc_v[pl.ds(kb + f, 16)] * inv
            return cc

        lax.fori_loop(0, 2, _row, 0)
        r0 = pl.multiple_of(sid * SPT + g * 2, 2)
        for c in range(2):
            @pl.when(cid == c)
            def _wr(c=c):
                dst = hmn0_hbm if c == 0 else hmn1_hbm
                pltpu.sync_copy(drn_v, dst.at[pl.ds(r0, 2)])
        return c

    lax.fori_loop(0, SPT // 2, _norm, 0)


def _edge_call(tgt, nb0, cur0, nb1, cur1, srctab, cs):
    mesh = plsc.VectorSubcoreMesh(core_axis_name="c", subcore_axis_name="s")
    f32 = jnp.float32
    kern = pl.kernel(
        _edge_body,
        out_type=[
            jax.ShapeDtypeStruct((B, 256), f32),
            jax.ShapeDtypeStruct((B, 256), f32),
            jax.ShapeDtypeStruct((B,), jnp.int32),
        ],
        mesh=mesh,
        compiler_params=pltpu.CompilerParams(needs_layout_passes=False),
        scratch_types=[
            pltpu.VMEM((NPAD,), jnp.int32),
            pltpu.VMEM((SPT * ACCW,), f32),
            pltpu.VMEM((PCAP,), jnp.int32),
            pltpu.VMEM((PCAP,), jnp.int32),
            pltpu.VMEM((FB, 128), f32),
            pltpu.VMEM((FB, 128), f32),
            pltpu.VMEM((STAGE,), jnp.int32),
            pltpu.VMEM((STAGE,), jnp.int32),
            pltpu.VMEM((64,), jnp.int32),
            pltpu.VMEM((64,), jnp.int32),
            pltpu.VMEM((32,), f32),
            pltpu.VMEM((2, 256), f32),
            pltpu.SemaphoreType.DMA,
        ],
    )
    return kern(tgt, nb0, cur0, nb1, cur1, srctab, cs)


def _gath_body(hmn0_hbm, hmn1_hbm, mrow_hbm, hm0, hm1, mr_v, buf_v, sem):
    cid = lax.axis_index("c")
    sid = lax.axis_index("s")
    wid = cid * 16 + sid
    base = pl.multiple_of(wid * (B // 32), B // 32)
    pltpu.sync_copy(mrow_hbm.at[pl.ds(base, B // 32)], mr_v)
    for m in range(M):
        src = hmn0_hbm if m == 0 else hmn1_hbm
        dst = hm0 if m == 0 else hm1
        pltpu.async_copy(src.at[mr_v], buf_v, sem).wait()
        pltpu.sync_copy(buf_v, dst.at[pl.ds(base, B // 32)])


def _gath_call(hmn0, hmn1, mrow):
    mesh = plsc.VectorSubcoreMesh(core_axis_name="c", subcore_axis_name="s")
    f32 = jnp.float32
    kern = pl.kernel(
        _gath_body,
        out_type=[
            jax.ShapeDtypeStruct((B, 256), f32),
            jax.ShapeDtypeStruct((B, 256), f32),
        ],
        mesh=mesh,
        compiler_params=pltpu.CompilerParams(needs_layout_passes=False),
        scratch_types=[
            pltpu.VMEM((B // 32,), jnp.int32),
            pltpu.VMEM((B // 32, 256), f32),
            pltpu.SemaphoreType.DMA,
        ],
    )
    return kern(hmn0, hmn1, mrow)


def _post_body(h0_ref, h1_ref, ws_ref, bs_ref, as_ref, wc_ref, bc_ref,
               log_ref, emb_ref):
    h0 = _lrelu(h0_ref[...])
    h1 = _lrelu(h1_ref[...])

    def att(h):
        s = jnp.tanh(
            lax.dot_general(h, ws_ref[...], (((1,), (1,)), ((), ())),
                            preferred_element_type=jnp.float32) + bs_ref[...])
        return jnp.mean(jnp.sum(as_ref[...] * s, axis=1))

    a0 = att(h0)
    a1 = att(h1)
    mx = jnp.maximum(a0, a1)
    e0 = jnp.exp(a0 - mx)
    e1 = jnp.exp(a1 - mx)
    b0 = e0 / (e0 + e1)
    b1 = e1 / (e0 + e1)
    emb = b0 * h0 + b1 * h1
    emb_ref[...] = emb
    log_ref[...] = lax.dot_general(emb, wc_ref[...], (((1,), (1,)), ((), ())),
                                   preferred_element_type=jnp.float32) + bc_ref[...]


def _post_call(h0, h1, wsem, bsem, asem, wcls, bcls):
    return pl.pallas_call(
        _post_body,
        out_shape=[
            jax.ShapeDtypeStruct((B, 16), jnp.float32),
            jax.ShapeDtypeStruct((B, H * D), jnp.float32),
        ],
    )(h0, h1, wsem, bsem, asem, wcls, bcls)


def kernel(target_nodes, metapath_list, node_type_mapping, node_feature_list,
           W_proj, b_proj, attn, W_sem, b_sem, a_sem, W_cls, b_cls):
    f32 = jnp.float32
    X = node_feature_list[0]
    Xp = jnp.pad(X, ((0, NPAD - N), (0, 0)))
    ntm = jnp.pad(node_type_mapping, (0, NPAD - N), constant_values=1)
    maskf = (ntm == 0).astype(f32)[:, None]
    a_r = attn.reshape(M, H, 2 * D)
    AW = jnp.concatenate(
        [a_r[0, :, :D].T, a_r[1, :, :D].T, a_r[0, :, D:].T, a_r[1, :, D:].T],
        axis=1)
    Wt = W_proj.T
    bp = b_proj[None, :]

    nb0 = metapath_list[0, :, 0]
    cur0 = metapath_list[0, :, 1]
    nb1 = metapath_list[1, :, 0]
    cur1 = metapath_list[1, :, 1]

    srctab, cs = _pre_call(Xp, Wt, AW, bp, maskf)

    hmn0, hmn1, mrow = _edge_call(target_nodes, nb0, cur0, nb1, cur1,
                                  srctab, cs)

    hm0, hm1 = _gath_call(hmn0, hmn1, mrow)

    logits, emb = _post_call(hm0, hm1, W_sem, b_sem[None, :], a_sem,
                             W_cls, b_cls[None, :])
    return (logits, emb)


# speedup vs baseline: 50.4029x; 1.0861x over previous
import jax
import jax.numpy as jnp
from jax import lax
from jax.experimental import pallas as pl
from jax.experimental.pallas import tpu as pltpu
from jax.experimental.pallas import tpu_sc as plsc

N = 10000
NPAD = 10240
DRAW = 128
D = 64
H = 4
M = 2
B = 4096
E = 160000
SLOPE = 0.01
STAGE = 1280
NSTG = E // STAGE
FB = 128
PCAP = STAGE + 2 * FB
SPT = B // 16
STW = 48
ACCW = 272
RB = 1024


def _lrelu(x):
    return jnp.where(x > 0, x, SLOPE * x)


def _pre_body(x_ref, wt_ref, aw_ref, bp_ref, mk_ref, st_ref, cs_ref):
    i = pl.program_id(0)
    nfb = (jnp.dot(x_ref[...], wt_ref[...], preferred_element_type=jnp.float32)
           + bp_ref[...]) * mk_ref[...]
    sc = jnp.dot(nfb, aw_ref[...], preferred_element_type=jnp.float32)
    st_ref[...] = jnp.concatenate(
        [nfb, sc[:, 8:16], sc[:, 0:8],
         jnp.zeros((RB, 128 - D - 16), jnp.float32)], axis=1)
    bm = jnp.max(sc, axis=0)

    @pl.when(i == 0)
    def _init():
        cs_ref[...] = jnp.full((1, 128), -1e30, jnp.float32)

    cs_ref[0:1, 0:16] = jnp.maximum(cs_ref[0:1, 0:16], bm[None, :])

    @pl.when(i == pl.num_programs(0) - 1)
    def _fin():
        v = cs_ref[0:1, 0:16]
        cs_ref[0:1, 16:24] = _lrelu(v[:, 0:8] + v[:, 8:16])


def _pre_call(xp, wt, aw, bp, maskf):
    ng = NPAD // RB
    return pl.pallas_call(
        _pre_body,
        grid=(ng,),
        in_specs=[
            pl.BlockSpec((RB, DRAW), lambda i: (i, 0)),
            pl.BlockSpec((DRAW, D), lambda i: (0, 0)),
            pl.BlockSpec((D, 16), lambda i: (0, 0)),
            pl.BlockSpec((1, D), lambda i: (0, 0)),
            pl.BlockSpec((RB, 1), lambda i: (i, 0)),
        ],
        out_specs=[
            pl.BlockSpec((RB, 128), lambda i: (i, 0)),
            pl.BlockSpec((1, 128), lambda i: (0, 0)),
        ],
        out_shape=[
            jax.ShapeDtypeStruct((NPAD, 128), jnp.float32),
            jax.ShapeDtypeStruct((1, 128), jnp.float32),
        ],
    )(xp, wt, aw, bp, maskf)


def _edge_body(tgt_hbm, nb0_hbm, cur0_hbm, nb1_hbm, cur1_hbm, st_hbm, cs_hbm,
               hmn0_hbm, hmn1_hbm, mrow_hbm,
               map_v, acc_v, pnb_v, pcur_v, rs_v, rd_v,
               nb_v, cur_v, tch_v, mch_v, cs_v, drn_v, sem, sem2):
    cid = lax.axis_index("c")
    sid = lax.axis_index("s")
    wid = cid * 16 + sid
    lanes = lax.iota(jnp.int32, 16)
    zpad = jnp.zeros((16,), jnp.float32)
    sent16 = jnp.full((16,), NPAD - 1, jnp.int32)

    pltpu.sync_copy(cs_hbm.at[0, pl.ds(0, 32)], cs_v)

    minus1 = jnp.full((16,), -1, jnp.int32)

    def _init_map(i, c):
        map_v[pl.ds(pl.multiple_of(i * 16, 16), 16)] = minus1
        return c

    lax.fori_loop(0, NPAD // 16, _init_map, 0)

    def _scat_map(j, c):
        pltpu.sync_copy(tgt_hbm.at[pl.ds(pl.multiple_of(j * 64, 64), 64)],
                        tch_v)
        for g in range(4):
            t16 = tch_v[pl.ds(g * 16, 16)]
            plsc.store_scatter(map_v, [t16], lanes + j * 64 + g * 16)
        return c

    lax.fori_loop(0, B // 64, _scat_map, 0)

    @pl.when(wid == 0)
    def _mrow():
        def body(j, c):
            jb = pl.multiple_of(j * 64, 64)
            pltpu.sync_copy(tgt_hbm.at[pl.ds(jb, 64)], tch_v)
            for g in range(4):
                t16 = tch_v[pl.ds(g * 16, 16)]
                mch_v[pl.ds(g * 16, 16)] = plsc.load_gather(map_v, [t16])
            pltpu.sync_copy(mch_v, mrow_hbm.at[pl.ds(jb, 64)])
            return c

        lax.fori_loop(0, B // 64, body, 0)

    def _zero_acc(i, c):
        acc_v[pl.ds(i * 16, 16)] = zpad
        return c

    lax.fori_loop(0, SPT * ACCW // 16, _zero_acc, 0)

    nb_hbm = (nb0_hbm, nb1_hbm)
    cur_hbm = (cur0_hbm, cur1_hbm)
    sid16 = jnp.zeros((16,), jnp.int32) + sid

    def _flush(b, cc):
        fb = pl.multiple_of(b * FB, FB)
        cp1 = pltpu.async_copy(st_hbm.at[pnb_v.at[pl.ds(fb, FB)]], rs_v, sem)
        cp2 = pltpu.async_copy(st_hbm.at[pcur_v.at[pl.ds(fb, FB)]], rd_v,
                               sem2)
        cp1.wait()
        cp2.wait()

        def _grp(g, gc):
            go = pl.multiple_of(g * 16, 16)
            cur16 = pcur_v[pl.ds(fb + go, 16)]
            m16 = plsc.load_gather(map_v, [cur16])
            own = jnp.right_shift(m16, 8) == sid16
            slot16 = jnp.bitwise_and(m16, 255)
            kvec = lanes + go
            ees = []
            for h in range(H):
                col_ss = jnp.zeros((16,), jnp.int32) + (cid * 4 + D + h)
                col_sd = jnp.zeros((16,), jnp.int32) + (cid * 4 + D + 8 + h)
                ss = plsc.load_gather(rs_v, [kvec, col_ss])
                sd = plsc.load_gather(rd_v, [kvec, col_sd])
                csp = plsc.load_gather(
                    cs_v, [jnp.zeros((16,), jnp.int32) + (cid * 4 + 16 + h)])
                ee = jnp.exp(_lrelu(sd + ss) - csp)
                ees.append(jnp.where(own, ee, 0.0))
            for l in range(16):
                k = go + l
                sb2 = slot16[l] * ACCW
                feats = [rs_v[k, pl.ds(j * 16, 16)] for j in range(4)]
                for h in range(H):
                    eh = ees[h][l]
                    for j in range(4):
                        colsl = pl.ds(sb2 + h * D + j * 16, 16)
                        acc_v[colsl] = acc_v[colsl] + eh * feats[j]
                dsl = pl.ds(sb2 + 256, 16)
                evl = [jnp.where(lanes == 0, ees[0][l], 0.0)]
                for h in range(1, H):
                    evl.append(jnp.where(lanes == h, ees[h][l],
                                         evl[h - 1]))
                acc_v[dsl] = acc_v[dsl] + evl[H - 1]
            return gc

        lax.fori_loop(0, FB // 16, _grp, 0)
        return cc

    def _stage(st, cnt):
        sb = pl.multiple_of(st * STAGE, STAGE)
        for c in range(2):
            @pl.when(cid == c)
            def _cp(c=c):
                pltpu.sync_copy(nb_hbm[c].at[pl.ds(sb, STAGE)], nb_v)
                pltpu.sync_copy(cur_hbm[c].at[pl.ds(sb, STAGE)], cur_v)

        def _scan(g, cn):
            go = pl.multiple_of(g * 16, 16)
            cur16 = cur_v[pl.ds(go, 16)]
            m16 = plsc.load_gather(map_v, [cur16])
            own = jnp.right_shift(m16, 8) == sid16
            csum = plsc.cumsum(jnp.where(own, jnp.int32(1), jnp.int32(0)))
            pos16 = cn + csum - 1
            plsc.store_scatter(pnb_v, [pos16], nb_v[pl.ds(go, 16)], mask=own)
            plsc.store_scatter(pcur_v, [pos16], cur16, mask=own)
            return cn + csum[15]

        cnt = lax.fori_loop(0, STAGE // 16, _scan, cnt)
        nfull = cnt // FB
        lax.fori_loop(0, nfull, _flush, 0)
        rb = pl.multiple_of(nfull * FB, FB)
        for g in range(FB // 16):
            tnb = pnb_v[pl.ds(rb + g * 16, 16)]
            tcur = pcur_v[pl.ds(rb + g * 16, 16)]
            pnb_v[pl.ds(g * 16, 16)] = tnb
            pcur_v[pl.ds(g * 16, 16)] = tcur
        return cnt - nfull * FB

    cnt = lax.fori_loop(0, NSTG, _stage, jnp.int32(0))
    for p in range(FB // 16):
        plsc.store_scatter(pnb_v, [cnt + p * 16 + lanes], sent16)
        plsc.store_scatter(pcur_v, [cnt + p * 16 + lanes], sent16)
    lax.fori_loop(0, (cnt + FB - 1) // FB, _flush, 0)

    def _norm(g, c):
        def _row(l, cc):
            kb = (g * 2 + l) * ACCW
            dvec = acc_v[pl.ds(kb + 256, 16)]
            ivec = jnp.where(dvec > 0, 1.0 / dvec, 0.0)
            for h in range(H):
                inv = ivec[h]
                for j in range(4):
                    f = h * D + j * 16
                    drn_v[l, pl.ds(f, 16)] = acc_v[pl.ds(kb + f, 16)] * inv
            return cc

        lax.fori_loop(0, 2, _row, 0)
        r0 = pl.multiple_of(sid * SPT + g * 2, 2)
        for c in range(2):
            @pl.when(cid == c)
            def _wr(c=c):
                dst = hmn0_hbm if c == 0 else hmn1_hbm
                pltpu.sync_copy(drn_v, dst.at[pl.ds(r0, 2)])
        return c

    lax.fori_loop(0, SPT // 2, _norm, 0)


def _edge_call(tgt, nb0, cur0, nb1, cur1, srctab, cs):
    mesh = plsc.VectorSubcoreMesh(core_axis_name="c", subcore_axis_name="s")
    f32 = jnp.float32
    kern = pl.kernel(
        _edge_body,
        out_type=[
            jax.ShapeDtypeStruct((B, 256), f32),
            jax.ShapeDtypeStruct((B, 256), f32),
            jax.ShapeDtypeStruct((B,), jnp.int32),
        ],
        mesh=mesh,
        compiler_params=pltpu.CompilerParams(needs_layout_passes=False),
        scratch_types=[
            pltpu.VMEM((NPAD,), jnp.int32),
            pltpu.VMEM((SPT * ACCW,), f32),
            pltpu.VMEM((PCAP,), jnp.int32),
            pltpu.VMEM((PCAP,), jnp.int32),
            pltpu.VMEM((FB, 128), f32),
            pltpu.VMEM((FB, 128), f32),
            pltpu.VMEM((STAGE,), jnp.int32),
            pltpu.VMEM((STAGE,), jnp.int32),
            pltpu.VMEM((64,), jnp.int32),
            pltpu.VMEM((64,), jnp.int32),
            pltpu.VMEM((32,), f32),
            pltpu.VMEM((2, 256), f32),
            pltpu.SemaphoreType.DMA,
            pltpu.SemaphoreType.DMA,
        ],
    )
    return kern(tgt, nb0, cur0, nb1, cur1, srctab, cs)


def _gath_body(hmn0_hbm, hmn1_hbm, mrow_hbm, hm0, hm1, mr_v, buf_v, sem):
    cid = lax.axis_index("c")
    sid = lax.axis_index("s")
    wid = cid * 16 + sid
    base = pl.multiple_of(wid * (B // 32), B // 32)
    pltpu.sync_copy(mrow_hbm.at[pl.ds(base, B // 32)], mr_v)
    for m in range(M):
        src = hmn0_hbm if m == 0 else hmn1_hbm
        dst = hm0 if m == 0 else hm1
        pltpu.async_copy(src.at[mr_v], buf_v, sem).wait()
        pltpu.sync_copy(buf_v, dst.at[pl.ds(base, B // 32)])


def _gath_call(hmn0, hmn1, mrow):
    mesh = plsc.VectorSubcoreMesh(core_axis_name="c", subcore_axis_name="s")
    f32 = jnp.float32
    kern = pl.kernel(
        _gath_body,
        out_type=[
            jax.ShapeDtypeStruct((B, 256), f32),
            jax.ShapeDtypeStruct((B, 256), f32),
        ],
        mesh=mesh,
        compiler_params=pltpu.CompilerParams(needs_layout_passes=False),
        scratch_types=[
            pltpu.VMEM((B // 32,), jnp.int32),
            pltpu.VMEM((B // 32, 256), f32),
            pltpu.SemaphoreType.DMA,
        ],
    )
    return kern(hmn0, hmn1, mrow)


def _post_body(h0_ref, h1_ref, ws_ref, bs_ref, as_ref, wc_ref, bc_ref,
               log_ref, emb_ref):
    h0 = _lrelu(h0_ref[...])
    h1 = _lrelu(h1_ref[...])

    def att(h):
        s = jnp.tanh(
            lax.dot_general(h, ws_ref[...], (((1,), (1,)), ((), ())),
                            preferred_element_type=jnp.float32) + bs_ref[...])
        return jnp.mean(jnp.sum(as_ref[...] * s, axis=1))

    a0 = att(h0)
    a1 = att(h1)
    mx = jnp.maximum(a0, a1)
    e0 = jnp.exp(a0 - mx)
    e1 = jnp.exp(a1 - mx)
    b0 = e0 / (e0 + e1)
    b1 = e1 / (e0 + e1)
    emb = b0 * h0 + b1 * h1
    emb_ref[...] = emb
    log_ref[...] = lax.dot_general(emb, wc_ref[...], (((1,), (1,)), ((), ())),
                                   preferred_element_type=jnp.float32) + bc_ref[...]


def _post_call(h0, h1, wsem, bsem, asem, wcls, bcls):
    return pl.pallas_call(
        _post_body,
        out_shape=[
            jax.ShapeDtypeStruct((B, 16), jnp.float32),
            jax.ShapeDtypeStruct((B, H * D), jnp.float32),
        ],
    )(h0, h1, wsem, bsem, asem, wcls, bcls)


def kernel(target_nodes, metapath_list, node_type_mapping, node_feature_list,
           W_proj, b_proj, attn, W_sem, b_sem, a_sem, W_cls, b_cls):
    f32 = jnp.float32
    X = node_feature_list[0]
    Xp = jnp.pad(X, ((0, NPAD - N), (0, 0)))
    ntm = jnp.pad(node_type_mapping, (0, NPAD - N), constant_values=1)
    maskf = (ntm == 0).astype(f32)[:, None]
    a_r = attn.reshape(M, H, 2 * D)
    AW = jnp.concatenate(
        [a_r[0, :, :D].T, a_r[1, :, :D].T, a_r[0, :, D:].T, a_r[1, :, D:].T],
        axis=1)
    Wt = W_proj.T
    bp = b_proj[None, :]

    nb0 = metapath_list[0, :, 0]
    cur0 = metapath_list[0, :, 1]
    nb1 = metapath_list[1, :, 0]
    cur1 = metapath_list[1, :, 1]

    srctab, cs = _pre_call(Xp, Wt, AW, bp, maskf)

    hmn0, hmn1, mrow = _edge_call(target_nodes, nb0, cur0, nb1, cur1,
                                  srctab, cs)

    hm0, hm1 = _gath_call(hmn0, hmn1, mrow)

    logits, emb = _post_call(hm0, hm1, W_sem, b_sem[None, :], a_sem,
                             W_cls, b_cls[None, :])
    return (logits, emb)


# speedup vs baseline: 62.5279x; 1.2406x over previous
import jax
import jax.numpy as jnp
from jax import lax
from jax.experimental import pallas as pl
from jax.experimental.pallas import tpu as pltpu
from jax.experimental.pallas import tpu_sc as plsc

N = 10000
NPAD = 10240
DRAW = 128
D = 64
H = 4
M = 2
B = 4096
E = 160000
SLOPE = 0.01
STAGE = 1600
NSTG = E // STAGE
FB = 128
PCAP = STAGE + 2 * FB
SPT = B // 16
STW = 48
ACCW = 272
RB = 1024


def _lrelu(x):
    return jnp.where(x > 0, x, SLOPE * x)


def _pre_body(x_ref, wt_ref, aw_ref, bp_ref, mk_ref, st_ref, cs_ref):
    i = pl.program_id(0)
    nfb = (jnp.dot(x_ref[...], wt_ref[...], preferred_element_type=jnp.float32)
           + bp_ref[...]) * mk_ref[...]
    sc = jnp.dot(nfb, aw_ref[...], preferred_element_type=jnp.float32)
    st_ref[...] = jnp.concatenate(
        [nfb, sc[:, 8:16], sc[:, 0:8],
         jnp.zeros((RB, 128 - D - 16), jnp.float32)], axis=1)
    bm = jnp.max(sc, axis=0)

    @pl.when(i == 0)
    def _init():
        cs_ref[...] = jnp.full((1, 128), -1e30, jnp.float32)

    cs_ref[0:1, 0:16] = jnp.maximum(cs_ref[0:1, 0:16], bm[None, :])

    @pl.when(i == pl.num_programs(0) - 1)
    def _fin():
        v = cs_ref[0:1, 0:16]
        cs_ref[0:1, 16:24] = _lrelu(v[:, 0:8] + v[:, 8:16])


def _pre_call(xp, wt, aw, bp, maskf):
    ng = NPAD // RB
    return pl.pallas_call(
        _pre_body,
        grid=(ng,),
        in_specs=[
            pl.BlockSpec((RB, DRAW), lambda i: (i, 0)),
            pl.BlockSpec((DRAW, D), lambda i: (0, 0)),
            pl.BlockSpec((D, 16), lambda i: (0, 0)),
            pl.BlockSpec((1, D), lambda i: (0, 0)),
            pl.BlockSpec((RB, 1), lambda i: (i, 0)),
        ],
        out_specs=[
            pl.BlockSpec((RB, 128), lambda i: (i, 0)),
            pl.BlockSpec((1, 128), lambda i: (0, 0)),
        ],
        out_shape=[
            jax.ShapeDtypeStruct((NPAD, 128), jnp.float32),
            jax.ShapeDtypeStruct((1, 128), jnp.float32),
        ],
    )(xp, wt, aw, bp, maskf)


def _edge_body(tgt_hbm, nb0_hbm, cur0_hbm, nb1_hbm, cur1_hbm, st_hbm, cs_hbm,
               hmn0_hbm, hmn1_hbm, mrow_hbm,
               map_v, acc_v, pnb_v, pcur_v, rs_v, rd_v,
               nba_v, cura_v, nbb_v, curb_v, tch_v, mch_v, cs_v, drn_v,
               sem, sem2, sema, semb):
    cid = lax.axis_index("c")
    sid = lax.axis_index("s")
    wid = cid * 16 + sid
    lanes = lax.iota(jnp.int32, 16)
    zpad = jnp.zeros((16,), jnp.float32)
    sent16 = jnp.full((16,), NPAD - 1, jnp.int32)

    pltpu.sync_copy(cs_hbm.at[0, pl.ds(0, 32)], cs_v)

    minus1 = jnp.full((16,), -1, jnp.int32)

    def _init_map(i, c):
        map_v[pl.ds(pl.multiple_of(i * 16, 16), 16)] = minus1
        return c

    lax.fori_loop(0, NPAD // 16, _init_map, 0)

    def _scat_map(j, c):
        pltpu.sync_copy(tgt_hbm.at[pl.ds(pl.multiple_of(j * 64, 64), 64)],
                        tch_v)
        for g in range(4):
            t16 = tch_v[pl.ds(g * 16, 16)]
            plsc.store_scatter(map_v, [t16], lanes + j * 64 + g * 16)
        return c

    lax.fori_loop(0, B // 64, _scat_map, 0)

    @pl.when(wid == 0)
    def _mrow():
        def body(j, c):
            jb = pl.multiple_of(j * 64, 64)
            pltpu.sync_copy(tgt_hbm.at[pl.ds(jb, 64)], tch_v)
            for g in range(4):
                t16 = tch_v[pl.ds(g * 16, 16)]
                mch_v[pl.ds(g * 16, 16)] = plsc.load_gather(map_v, [t16])
            pltpu.sync_copy(mch_v, mrow_hbm.at[pl.ds(jb, 64)])
            return c

        lax.fori_loop(0, B // 64, body, 0)

    def _zero_acc(i, c):
        acc_v[pl.ds(i * 16, 16)] = zpad
        return c

    lax.fori_loop(0, SPT * ACCW // 16, _zero_acc, 0)

    nb_hbm = (nb0_hbm, nb1_hbm)
    cur_hbm = (cur0_hbm, cur1_hbm)
    sid16 = jnp.zeros((16,), jnp.int32) + sid

    def _flush(b, cc):
        fb = pl.multiple_of(b * FB, FB)
        cp1 = pltpu.async_copy(st_hbm.at[pnb_v.at[pl.ds(fb, FB)]], rs_v, sem)
        cp2 = pltpu.async_copy(st_hbm.at[pcur_v.at[pl.ds(fb, FB)]], rd_v,
                               sem2)
        cp1.wait()
        cp2.wait()

        def _grp(g, gc):
            go = pl.multiple_of(g * 16, 16)
            cur16 = pcur_v[pl.ds(fb + go, 16)]
            m16 = plsc.load_gather(map_v, [cur16])
            own = jnp.right_shift(m16, 8) == sid16
            slot16 = jnp.bitwise_and(m16, 255)
            kvec = lanes + go
            ees = []
            for h in range(H):
                col_ss = jnp.zeros((16,), jnp.int32) + (cid * 4 + D + h)
                col_sd = jnp.zeros((16,), jnp.int32) + (cid * 4 + D + 8 + h)
                ss = plsc.load_gather(rs_v, [kvec, col_ss])
                sd = plsc.load_gather(rd_v, [kvec, col_sd])
                csp = plsc.load_gather(
                    cs_v, [jnp.zeros((16,), jnp.int32) + (cid * 4 + 16 + h)])
                ee = jnp.exp(_lrelu(sd + ss) - csp)
                ees.append(jnp.where(own, ee, 0.0))
            for l in range(16):
                k = go + l
                sb2 = slot16[l] * ACCW
                feats = [rs_v[k, pl.ds(j * 16, 16)] for j in range(4)]
                for h in range(H):
                    eh = ees[h][l]
                    for j in range(4):
                        colsl = pl.ds(sb2 + h * D + j * 16, 16)
                        acc_v[colsl] = acc_v[colsl] + eh * feats[j]
                dsl = pl.ds(sb2 + 256, 16)
                evl = [jnp.where(lanes == 0, ees[0][l], 0.0)]
                for h in range(1, H):
                    evl.append(jnp.where(lanes == h, ees[h][l],
                                         evl[h - 1]))
                acc_v[dsl] = acc_v[dsl] + evl[H - 1]
            return gc

        lax.fori_loop(0, FB // 16, _grp, 0)
        return cc

    def _issue(sb, nbdst, curdst, s1):
        for c in range(2):
            @pl.when(cid == c)
            def _cp(c=c):
                pltpu.async_copy(nb_hbm[c].at[pl.ds(sb, STAGE)], nbdst, s1)
                pltpu.async_copy(cur_hbm[c].at[pl.ds(sb, STAGE)], curdst, s1)

    def _await(nbdst, curdst, s1):
        pltpu.make_async_copy(nb0_hbm.at[pl.ds(0, STAGE)], nbdst, s1).wait()
        pltpu.make_async_copy(cur0_hbm.at[pl.ds(0, STAGE)], curdst, s1).wait()

    def _scan_stage(nbuf, cbuf, cnt):
        def _scan(g, cn):
            go = pl.multiple_of(g * 16, 16)
            cur16 = cbuf[pl.ds(go, 16)]
            m16 = plsc.load_gather(map_v, [cur16])
            own = jnp.right_shift(m16, 8) == sid16
            csum = plsc.cumsum(jnp.where(own, jnp.int32(1), jnp.int32(0)))
            pos16 = cn + csum - 1
            plsc.store_scatter(pnb_v, [pos16], nbuf[pl.ds(go, 16)], mask=own)
            plsc.store_scatter(pcur_v, [pos16], cur16, mask=own)
            return cn + csum[15]

        cnt = lax.fori_loop(0, STAGE // 16, _scan, cnt)
        nfull = cnt // FB
        lax.fori_loop(0, nfull, _flush, 0)
        rb = pl.multiple_of(nfull * FB, FB)
        for g in range(FB // 16):
            tnb = pnb_v[pl.ds(rb + g * 16, 16)]
            tcur = pcur_v[pl.ds(rb + g * 16, 16)]
            pnb_v[pl.ds(g * 16, 16)] = tnb
            pcur_v[pl.ds(g * 16, 16)] = tcur
        return cnt - nfull * FB

    _issue(pl.multiple_of(0, STAGE), nba_v, cura_v, sema)

    def _pair(p, cnt):
        st2 = pl.multiple_of(2 * p * STAGE, STAGE)
        _issue(st2 + STAGE, nbb_v, curb_v, semb)
        _await(nba_v, cura_v, sema)
        cnt = _scan_stage(nba_v, cura_v, cnt)
        nxt = pl.multiple_of((st2 + 2 * STAGE) % (NSTG * STAGE), STAGE)
        _issue(nxt, nba_v, cura_v, sema)
        _await(nbb_v, curb_v, semb)
        cnt = _scan_stage(nbb_v, curb_v, cnt)
        return cnt

    cnt = lax.fori_loop(0, NSTG // 2, _pair, jnp.int32(0))
    _await(nba_v, cura_v, sema)
    for p in range(FB // 16):
        plsc.store_scatter(pnb_v, [cnt + p * 16 + lanes], sent16)
        plsc.store_scatter(pcur_v, [cnt + p * 16 + lanes], sent16)
    lax.fori_loop(0, (cnt + FB - 1) // FB, _flush, 0)

    def _norm(g, c):
        def _row(l, cc):
            kb = (g * 8 + l) * ACCW
            dvec = acc_v[pl.ds(kb + 256, 16)]
            ivec = jnp.where(dvec > 0, 1.0 / dvec, 0.0)
            for h in range(H):
                inv = ivec[h]
                for j in range(4):
                    f = h * D + j * 16
                    drn_v[l, pl.ds(f, 16)] = acc_v[pl.ds(kb + f, 16)] * inv
            return cc

        lax.fori_loop(0, 8, _row, 0)
        r0 = pl.multiple_of(sid * SPT + g * 8, 8)
        for c in range(2):
            @pl.when(cid == c)
            def _wr(c=c):
                dst = hmn0_hbm if c == 0 else hmn1_hbm
                pltpu.sync_copy(drn_v, dst.at[pl.ds(r0, 8)])
        return c

    lax.fori_loop(0, SPT // 8, _norm, 0)


def _edge_call(tgt, nb0, cur0, nb1, cur1, srctab, cs):
    mesh = plsc.VectorSubcoreMesh(core_axis_name="c", subcore_axis_name="s")
    f32 = jnp.float32
    kern = pl.kernel(
        _edge_body,
        out_type=[
            jax.ShapeDtypeStruct((B, 256), f32),
            jax.ShapeDtypeStruct((B, 256), f32),
            jax.ShapeDtypeStruct((B,), jnp.int32),
        ],
        mesh=mesh,
        compiler_params=pltpu.CompilerParams(needs_layout_passes=False),
        scratch_types=[
            pltpu.VMEM((NPAD,), jnp.int32),
            pltpu.VMEM((SPT * ACCW,), f32),
            pltpu.VMEM((PCAP,), jnp.int32),
            pltpu.VMEM((PCAP,), jnp.int32),
            pltpu.VMEM((FB, 128), f32),
            pltpu.VMEM((FB, 128), f32),
            pltpu.VMEM((STAGE,), jnp.int32),
            pltpu.VMEM((STAGE,), jnp.int32),
            pltpu.VMEM((STAGE,), jnp.int32),
            pltpu.VMEM((STAGE,), jnp.int32),
            pltpu.VMEM((64,), jnp.int32),
            pltpu.VMEM((64,), jnp.int32),
            pltpu.VMEM((32,), f32),
            pltpu.VMEM((8, 256), f32),
            pltpu.SemaphoreType.DMA,
            pltpu.SemaphoreType.DMA,
            pltpu.SemaphoreType.DMA,
            pltpu.SemaphoreType.DMA,
        ],
    )
    return kern(tgt, nb0, cur0, nb1, cur1, srctab, cs)


def _gath_body(hmn0_hbm, hmn1_hbm, mrow_hbm, hm0, hm1, mr_v, buf_v, sem):
    cid = lax.axis_index("c")
    sid = lax.axis_index("s")
    wid = cid * 16 + sid
    base = pl.multiple_of(wid * (B // 32), B // 32)
    pltpu.sync_copy(mrow_hbm.at[pl.ds(base, B // 32)], mr_v)
    for m in range(M):
        src = hmn0_hbm if m == 0 else hmn1_hbm
        dst = hm0 if m == 0 else hm1
        pltpu.async_copy(src.at[mr_v], buf_v, sem).wait()
        pltpu.sync_copy(buf_v, dst.at[pl.ds(base, B // 32)])


def _gath_call(hmn0, hmn1, mrow):
    mesh = plsc.VectorSubcoreMesh(core_axis_name="c", subcore_axis_name="s")
    f32 = jnp.float32
    kern = pl.kernel(
        _gath_body,
        out_type=[
            jax.ShapeDtypeStruct((B, 256), f32),
            jax.ShapeDtypeStruct((B, 256), f32),
        ],
        mesh=mesh,
        compiler_params=pltpu.CompilerParams(needs_layout_passes=False),
        scratch_types=[
            pltpu.VMEM((B // 32,), jnp.int32),
            pltpu.VMEM((B // 32, 256), f32),
            pltpu.SemaphoreType.DMA,
        ],
    )
    return kern(hmn0, hmn1, mrow)


def _post_body(h0_ref, h1_ref, ws_ref, bs_ref, as_ref, wc_ref, bc_ref,
               log_ref, emb_ref):
    h0 = _lrelu(h0_ref[...])
    h1 = _lrelu(h1_ref[...])

    def att(h):
        s = jnp.tanh(
            lax.dot_general(h, ws_ref[...], (((1,), (1,)), ((), ())),
                            preferred_element_type=jnp.float32) + bs_ref[...])
        return jnp.mean(jnp.sum(as_ref[...] * s, axis=1))

    a0 = att(h0)
    a1 = att(h1)
    mx = jnp.maximum(a0, a1)
    e0 = jnp.exp(a0 - mx)
    e1 = jnp.exp(a1 - mx)
    b0 = e0 / (e0 + e1)
    b1 = e1 / (e0 + e1)
    emb = b0 * h0 + b1 * h1
    emb_ref[...] = emb
    log_ref[...] = lax.dot_general(emb, wc_ref[...], (((1,), (1,)), ((), ())),
                                   preferred_element_type=jnp.float32) + bc_ref[...]


def _post_call(h0, h1, wsem, bsem, asem, wcls, bcls):
    return pl.pallas_call(
        _post_body,
        out_shape=[
            jax.ShapeDtypeStruct((B, 16), jnp.float32),
            jax.ShapeDtypeStruct((B, H * D), jnp.float32),
        ],
    )(h0, h1, wsem, bsem, asem, wcls, bcls)


def kernel(target_nodes, metapath_list, node_type_mapping, node_feature_list,
           W_proj, b_proj, attn, W_sem, b_sem, a_sem, W_cls, b_cls):
    f32 = jnp.float32
    X = node_feature_list[0]
    Xp = jnp.pad(X, ((0, NPAD - N), (0, 0)))
    ntm = jnp.pad(node_type_mapping, (0, NPAD - N), constant_values=1)
    maskf = (ntm == 0).astype(f32)[:, None]
    a_r = attn.reshape(M, H, 2 * D)
    AW = jnp.concatenate(
        [a_r[0, :, :D].T, a_r[1, :, :D].T, a_r[0, :, D:].T, a_r[1, :, D:].T],
        axis=1)
    Wt = W_proj.T
    bp = b_proj[None, :]

    nb0 = metapath_list[0, :, 0]
    cur0 = metapath_list[0, :, 1]
    nb1 = metapath_list[1, :, 0]
    cur1 = metapath_list[1, :, 1]

    srctab, cs = _pre_call(Xp, Wt, AW, bp, maskf)

    hmn0, hmn1, mrow = _edge_call(target_nodes, nb0, cur0, nb1, cur1,
                                  srctab, cs)

    hm0, hm1 = _gath_call(hmn0, hmn1, mrow)

    logits, emb = _post_call(hm0, hm1, W_sem, b_sem[None, :], a_sem,
                             W_cls, b_cls[None, :])
    return (logits, emb)


# speedup vs baseline: 66.8591x; 1.0693x over previous
import jax
import jax.numpy as jnp
from jax import lax
from jax.experimental import pallas as pl
from jax.experimental.pallas import tpu as pltpu
from jax.experimental.pallas import tpu_sc as plsc

N = 10000
NPAD = 10240
DRAW = 128
D = 64
H = 4
M = 2
B = 4096
E = 160000
SLOPE = 0.01
NW = 32
STAGE = 1600
NSTG = E // STAGE
FB = 128
PCAP = STAGE + 2 * FB
SPT = B // 16
STW = 48
ACCW = 272
RB = 1024


def _lrelu(x):
    return jnp.where(x > 0, x, SLOPE * x)


def _pre_body(x_ref, wt_ref, aw_ref, bp_ref, mk_ref, st_ref, cs_ref):
    i = pl.program_id(0)
    nfb = (jnp.dot(x_ref[...], wt_ref[...], preferred_element_type=jnp.float32)
           + bp_ref[...]) * mk_ref[...]
    sc = jnp.dot(nfb, aw_ref[...], preferred_element_type=jnp.float32)
    st_ref[...] = jnp.concatenate(
        [nfb, sc[:, 8:16], sc[:, 0:8],
         jnp.zeros((RB, 128 - D - 16), jnp.float32)], axis=1)
    bm = jnp.max(sc, axis=0)

    @pl.when(i == 0)
    def _init():
        cs_ref[...] = jnp.full((1, 128), -1e30, jnp.float32)

    cs_ref[0:1, 0:16] = jnp.maximum(cs_ref[0:1, 0:16], bm[None, :])

    @pl.when(i == pl.num_programs(0) - 1)
    def _fin():
        v = cs_ref[0:1, 0:16]
        cs_ref[0:1, 16:24] = _lrelu(v[:, 0:8] + v[:, 8:16])


def _pre_call(xp, wt, aw, bp, maskf):
    ng = NPAD // RB
    return pl.pallas_call(
        _pre_body,
        grid=(ng,),
        in_specs=[
            pl.BlockSpec((RB, DRAW), lambda i: (i, 0)),
            pl.BlockSpec((DRAW, D), lambda i: (0, 0)),
            pl.BlockSpec((D, 16), lambda i: (0, 0)),
            pl.BlockSpec((1, D), lambda i: (0, 0)),
            pl.BlockSpec((RB, 1), lambda i: (i, 0)),
        ],
        out_specs=[
            pl.BlockSpec((RB, 128), lambda i: (i, 0)),
            pl.BlockSpec((1, 128), lambda i: (0, 0)),
        ],
        out_shape=[
            jax.ShapeDtypeStruct((NPAD, 128), jnp.float32),
            jax.ShapeDtypeStruct((1, 128), jnp.float32),
        ],
    )(xp, wt, aw, bp, maskf)


def _edge_body(tgt_hbm, nb0_hbm, cur0_hbm, nb1_hbm, cur1_hbm, st_hbm, cs_hbm,
               hmn0_hbm, hmn1_hbm, mrow_hbm,
               map_v, acc_v, pnb_v, pcur_v, rs_v, rd_v,
               nba_v, cura_v, nbb_v, curb_v, tch_v, mch_v, cs_v, drn_v,
               sem, sem2, sema, semb):
    cid = lax.axis_index("c")
    sid = lax.axis_index("s")
    wid = cid * 16 + sid
    lanes = lax.iota(jnp.int32, 16)
    zpad = jnp.zeros((16,), jnp.float32)
    sent16 = jnp.full((16,), NPAD - 1, jnp.int32)

    pltpu.sync_copy(cs_hbm.at[0, pl.ds(0, 32)], cs_v)

    minus1 = jnp.full((16,), -1, jnp.int32)

    def _init_map(i, c):
        map_v[pl.ds(pl.multiple_of(i * 16, 16), 16)] = minus1
        return c

    lax.fori_loop(0, NPAD // 16, _init_map, 0)

    def _scat_map(j, c):
        pltpu.sync_copy(tgt_hbm.at[pl.ds(pl.multiple_of(j * 64, 64), 64)],
                        tch_v)
        for g in range(4):
            t16 = tch_v[pl.ds(g * 16, 16)]
            plsc.store_scatter(map_v, [t16], lanes + j * 64 + g * 16)
        return c

    lax.fori_loop(0, B // 64, _scat_map, 0)

    def _mrow(j, c):
        jb = pl.multiple_of(wid * (B // NW) + j * 64, 64)
        pltpu.sync_copy(tgt_hbm.at[pl.ds(jb, 64)], tch_v)
        for g in range(4):
            t16 = tch_v[pl.ds(g * 16, 16)]
            mch_v[pl.ds(g * 16, 16)] = plsc.load_gather(map_v, [t16])
        pltpu.sync_copy(mch_v, mrow_hbm.at[pl.ds(jb, 64)])
        return c

    lax.fori_loop(0, B // NW // 64, _mrow, 0)

    def _zero_acc(i, c):
        acc_v[pl.ds(i * 16, 16)] = zpad
        return c

    lax.fori_loop(0, SPT * ACCW // 16, _zero_acc, 0)

    nb_hbm = (nb0_hbm, nb1_hbm)
    cur_hbm = (cur0_hbm, cur1_hbm)
    sid16 = jnp.zeros((16,), jnp.int32) + sid

    def _flush(b, cc):
        fb = pl.multiple_of(b * FB, FB)
        cp1 = pltpu.async_copy(st_hbm.at[pnb_v.at[pl.ds(fb, FB)]], rs_v, sem)
        cp2 = pltpu.async_copy(st_hbm.at[pcur_v.at[pl.ds(fb, FB)]], rd_v,
                               sem2)
        cp1.wait()
        cp2.wait()

        def _grp(g, gc):
            go = pl.multiple_of(g * 16, 16)
            cur16 = pcur_v[pl.ds(fb + go, 16)]
            m16 = plsc.load_gather(map_v, [cur16])
            own = jnp.right_shift(m16, 8) == sid16
            slot16 = jnp.bitwise_and(m16, 255)
            kvec = lanes + go
            ees = []
            for h in range(H):
                col_ss = jnp.zeros((16,), jnp.int32) + (cid * 4 + D + h)
                col_sd = jnp.zeros((16,), jnp.int32) + (cid * 4 + D + 8 + h)
                ss = plsc.load_gather(rs_v, [kvec, col_ss])
                sd = plsc.load_gather(rd_v, [kvec, col_sd])
                csp = plsc.load_gather(
                    cs_v, [jnp.zeros((16,), jnp.int32) + (cid * 4 + 16 + h)])
                ee = jnp.exp(_lrelu(sd + ss) - csp)
                ees.append(jnp.where(own, ee, 0.0))
            for l in range(16):
                k = go + l
                sb2 = slot16[l] * ACCW
                feats = [rs_v[k, pl.ds(j * 16, 16)] for j in range(4)]
                for h in range(H):
                    eh = ees[h][l]
                    for j in range(4):
                        colsl = pl.ds(sb2 + h * D + j * 16, 16)
                        acc_v[colsl] = acc_v[colsl] + eh * feats[j]
                dsl = pl.ds(sb2 + 256, 16)
                evl = [jnp.where(lanes == 0, ees[0][l], 0.0)]
                for h in range(1, H):
                    evl.append(jnp.where(lanes == h, ees[h][l],
                                         evl[h - 1]))
                acc_v[dsl] = acc_v[dsl] + evl[H - 1]
            return gc

        lax.fori_loop(0, FB // 16, _grp, 0)
        return cc

    def _issue(sb, nbdst, curdst, s1):
        for c in range(2):
            @pl.when(cid == c)
            def _cp(c=c):
                pltpu.async_copy(nb_hbm[c].at[pl.ds(sb, STAGE)], nbdst, s1)
                pltpu.async_copy(cur_hbm[c].at[pl.ds(sb, STAGE)], curdst, s1)

    def _await(nbdst, curdst, s1):
        pltpu.make_async_copy(nb0_hbm.at[pl.ds(0, STAGE)], nbdst, s1).wait()
        pltpu.make_async_copy(cur0_hbm.at[pl.ds(0, STAGE)], curdst, s1).wait()

    def _scan_stage(nbuf, cbuf, cnt):
        def _scan(g, cn):
            go = pl.multiple_of(g * 16, 16)
            cur16 = cbuf[pl.ds(go, 16)]
            m16 = plsc.load_gather(map_v, [cur16])
            own = jnp.right_shift(m16, 8) == sid16
            plsc.store_compressed(pnb_v.at[pl.ds(cn, 16)],
                                  nbuf[pl.ds(go, 16)], mask=own)
            plsc.store_compressed(pcur_v.at[pl.ds(cn, 16)], cur16, mask=own)
            npop = plsc.all_reduce_population_count(own)
            return cn + npop[0]

        cnt = lax.fori_loop(0, STAGE // 16, _scan, cnt)
        nfull = cnt // FB
        lax.fori_loop(0, nfull, _flush, 0)
        rb = pl.multiple_of(nfull * FB, FB)
        for g in range(FB // 16):
            tnb = pnb_v[pl.ds(rb + g * 16, 16)]
            tcur = pcur_v[pl.ds(rb + g * 16, 16)]
            pnb_v[pl.ds(g * 16, 16)] = tnb
            pcur_v[pl.ds(g * 16, 16)] = tcur
        return cnt - nfull * FB

    _issue(pl.multiple_of(0, STAGE), nba_v, cura_v, sema)

    def _pair(p, cnt):
        st2 = pl.multiple_of(2 * p * STAGE, STAGE)
        _issue(st2 + STAGE, nbb_v, curb_v, semb)
        _await(nba_v, cura_v, sema)
        cnt = _scan_stage(nba_v, cura_v, cnt)
        nxt = pl.multiple_of((st2 + 2 * STAGE) % (NSTG * STAGE), STAGE)
        _issue(nxt, nba_v, cura_v, sema)
        _await(nbb_v, curb_v, semb)
        cnt = _scan_stage(nbb_v, curb_v, cnt)
        return cnt

    cnt = lax.fori_loop(0, NSTG // 2, _pair, jnp.int32(0))
    _await(nba_v, cura_v, sema)
    for p in range(FB // 16):
        plsc.store_scatter(pnb_v, [cnt + p * 16 + lanes], sent16)
        plsc.store_scatter(pcur_v, [cnt + p * 16 + lanes], sent16)
    lax.fori_loop(0, (cnt + FB - 1) // FB, _flush, 0)

    def _norm(g, c):
        def _row(l, cc):
            kb = (g * 8 + l) * ACCW
            dvec = acc_v[pl.ds(kb + 256, 16)]
            ivec = jnp.where(dvec > 0, 1.0 / dvec, 0.0)
            for h in range(H):
                inv = ivec[h]
                for j in range(4):
                    f = h * D + j * 16
                    drn_v[l, pl.ds(f, 16)] = acc_v[pl.ds(kb + f, 16)] * inv
            return cc

        lax.fori_loop(0, 8, _row, 0)
        r0 = pl.multiple_of(sid * SPT + g * 8, 8)
        for c in range(2):
            @pl.when(cid == c)
            def _wr(c=c):
                dst = hmn0_hbm if c == 0 else hmn1_hbm
                pltpu.sync_copy(drn_v, dst.at[pl.ds(r0, 8)])
        return c

    lax.fori_loop(0, SPT // 8, _norm, 0)


def _edge_call(tgt, nb0, cur0, nb1, cur1, srctab, cs):
    mesh = plsc.VectorSubcoreMesh(core_axis_name="c", subcore_axis_name="s")
    f32 = jnp.float32
    kern = pl.kernel(
        _edge_body,
        out_type=[
            jax.ShapeDtypeStruct((B, 256), f32),
            jax.ShapeDtypeStruct((B, 256), f32),
            jax.ShapeDtypeStruct((B,), jnp.int32),
        ],
        mesh=mesh,
        compiler_params=pltpu.CompilerParams(needs_layout_passes=False),
        scratch_types=[
            pltpu.VMEM((NPAD,), jnp.int32),
            pltpu.VMEM((SPT * ACCW,), f32),
            pltpu.VMEM((PCAP,), jnp.int32),
            pltpu.VMEM((PCAP,), jnp.int32),
            pltpu.VMEM((FB, 128), f32),
            pltpu.VMEM((FB, 128), f32),
            pltpu.VMEM((STAGE,), jnp.int32),
            pltpu.VMEM((STAGE,), jnp.int32),
            pltpu.VMEM((STAGE,), jnp.int32),
            pltpu.VMEM((STAGE,), jnp.int32),
            pltpu.VMEM((64,), jnp.int32),
            pltpu.VMEM((64,), jnp.int32),
            pltpu.VMEM((32,), f32),
            pltpu.VMEM((8, 256), f32),
            pltpu.SemaphoreType.DMA,
            pltpu.SemaphoreType.DMA,
            pltpu.SemaphoreType.DMA,
            pltpu.SemaphoreType.DMA,
        ],
    )
    return kern(tgt, nb0, cur0, nb1, cur1, srctab, cs)


def _gath_body(hmn0_hbm, hmn1_hbm, mrow_hbm, hm0, hm1, mr_v, buf_v, sem):
    cid = lax.axis_index("c")
    sid = lax.axis_index("s")
    wid = cid * 16 + sid
    base = pl.multiple_of(wid * (B // 32), B // 32)
    pltpu.sync_copy(mrow_hbm.at[pl.ds(base, B // 32)], mr_v)
    for m in range(M):
        src = hmn0_hbm if m == 0 else hmn1_hbm
        dst = hm0 if m == 0 else hm1
        pltpu.async_copy(src.at[mr_v], buf_v, sem).wait()
        pltpu.sync_copy(buf_v, dst.at[pl.ds(base, B // 32)])


def _gath_call(hmn0, hmn1, mrow):
    mesh = plsc.VectorSubcoreMesh(core_axis_name="c", subcore_axis_name="s")
    f32 = jnp.float32
    kern = pl.kernel(
        _gath_body,
        out_type=[
            jax.ShapeDtypeStruct((B, 256), f32),
            jax.ShapeDtypeStruct((B, 256), f32),
        ],
        mesh=mesh,
        compiler_params=pltpu.CompilerParams(needs_layout_passes=False),
        scratch_types=[
            pltpu.VMEM((B // 32,), jnp.int32),
            pltpu.VMEM((B // 32, 256), f32),
            pltpu.SemaphoreType.DMA,
        ],
    )
    return kern(hmn0, hmn1, mrow)


def _post_body(h0_ref, h1_ref, ws_ref, bs_ref, as_ref, wc_ref, bc_ref,
               log_ref, emb_ref):
    h0 = _lrelu(h0_ref[...])
    h1 = _lrelu(h1_ref[...])

    def att(h):
        s = jnp.tanh(
            lax.dot_general(h, ws_ref[...], (((1,), (1,)), ((), ())),
                            preferred_element_type=jnp.float32) + bs_ref[...])
        return jnp.mean(jnp.sum(as_ref[...] * s, axis=1))

    a0 = att(h0)
    a1 = att(h1)
    mx = jnp.maximum(a0, a1)
    e0 = jnp.exp(a0 - mx)
    e1 = jnp.exp(a1 - mx)
    b0 = e0 / (e0 + e1)
    b1 = e1 / (e0 + e1)
    emb = b0 * h0 + b1 * h1
    emb_ref[...] = emb
    log_ref[...] = lax.dot_general(emb, wc_ref[...], (((1,), (1,)), ((), ())),
                                   preferred_element_type=jnp.float32) + bc_ref[...]


def _post_call(h0, h1, wsem, bsem, asem, wcls, bcls):
    return pl.pallas_call(
        _post_body,
        out_shape=[
            jax.ShapeDtypeStruct((B, 16), jnp.float32),
            jax.ShapeDtypeStruct((B, H * D), jnp.float32),
        ],
    )(h0, h1, wsem, bsem, asem, wcls, bcls)


def kernel(target_nodes, metapath_list, node_type_mapping, node_feature_list,
           W_proj, b_proj, attn, W_sem, b_sem, a_sem, W_cls, b_cls):
    f32 = jnp.float32
    X = node_feature_list[0]
    Xp = jnp.pad(X, ((0, NPAD - N), (0, 0)))
    ntm = jnp.pad(node_type_mapping, (0, NPAD - N), constant_values=1)
    maskf = (ntm == 0).astype(f32)[:, None]
    a_r = attn.reshape(M, H, 2 * D)
    AW = jnp.concatenate(
        [a_r[0, :, :D].T, a_r[1, :, :D].T, a_r[0, :, D:].T, a_r[1, :, D:].T],
        axis=1)
    Wt = W_proj.T
    bp = b_proj[None, :]

    nb0 = metapath_list[0, :, 0]
    cur0 = metapath_list[0, :, 1]
    nb1 = metapath_list[1, :, 0]
    cur1 = metapath_list[1, :, 1]

    srctab, cs = _pre_call(Xp, Wt, AW, bp, maskf)

    hmn0, hmn1, mrow = _edge_call(target_nodes, nb0, cur0, nb1, cur1,
                                  srctab, cs)

    hm0, hm1 = _gath_call(hmn0, hmn1, mrow)

    logits, emb = _post_call(hm0, hm1, W_sem, b_sem[None, :], a_sem,
                             W_cls, b_cls[None, :])
    return (logits, emb)


# speedup vs baseline: 79.6353x; 1.1911x over previous
import jax
import jax.numpy as jnp
from jax import lax
from jax.experimental import pallas as pl
from jax.experimental.pallas import tpu as pltpu
from jax.experimental.pallas import tpu_sc as plsc

N = 10000
NPAD = 10240
DRAW = 128
D = 64
H = 4
M = 2
B = 4096
E = 160000
SLOPE = 0.01
NW = 32
STAGE = 1600
NSTG = E // STAGE
FB = 128
PCAP = STAGE + 2 * FB
SPT = B // 16
STW = 48
ACCW = 272
RB = 1024


def _lrelu(x):
    return jnp.where(x > 0, x, SLOPE * x)


def _pre_body(x_ref, wt_ref, aw_ref, bp_ref, mk_ref, st_ref, cs_ref):
    i = pl.program_id(0)
    nfb = (jnp.dot(x_ref[...], wt_ref[...], preferred_element_type=jnp.float32)
           + bp_ref[...]) * mk_ref[...]
    sc = jnp.dot(nfb, aw_ref[...], preferred_element_type=jnp.float32)
    st_ref[...] = jnp.concatenate(
        [nfb, sc[:, 8:16], sc[:, 0:8],
         jnp.zeros((RB, 128 - D - 16), jnp.float32)], axis=1)
    bm = jnp.max(sc, axis=0)

    @pl.when(i == 0)
    def _init():
        cs_ref[...] = jnp.full((1, 128), -1e30, jnp.float32)

    cs_ref[0:1, 0:16] = jnp.maximum(cs_ref[0:1, 0:16], bm[None, :])

    @pl.when(i == pl.num_programs(0) - 1)
    def _fin():
        v = cs_ref[0:1, 0:16]
        cs_ref[0:1, 16:24] = _lrelu(v[:, 0:8] + v[:, 8:16])


def _pre_call(xp, wt, aw, bp, maskf):
    ng = NPAD // RB
    return pl.pallas_call(
        _pre_body,
        grid=(ng,),
        in_specs=[
            pl.BlockSpec((RB, DRAW), lambda i: (i, 0)),
            pl.BlockSpec((DRAW, D), lambda i: (0, 0)),
            pl.BlockSpec((D, 16), lambda i: (0, 0)),
            pl.BlockSpec((1, D), lambda i: (0, 0)),
            pl.BlockSpec((RB, 1), lambda i: (i, 0)),
        ],
        out_specs=[
            pl.BlockSpec((RB, 128), lambda i: (i, 0)),
            pl.BlockSpec((1, 128), lambda i: (0, 0)),
        ],
        out_shape=[
            jax.ShapeDtypeStruct((NPAD, 128), jnp.float32),
            jax.ShapeDtypeStruct((1, 128), jnp.float32),
        ],
    )(xp, wt, aw, bp, maskf)


def _edge_body(tgt_hbm, nb0_hbm, cur0_hbm, nb1_hbm, cur1_hbm, st_hbm, cs_hbm,
               hmn0_hbm, hmn1_hbm, mrow_hbm,
               map_v, acc_v, pnb_v, pcur_v, rs_v, rd_v,
               nba_v, cura_v, nbb_v, curb_v, tch_v, mch_v, cs_v, drn_v,
               sem, sem2, sema, semb):
    cid = lax.axis_index("c")
    sid = lax.axis_index("s")
    wid = cid * 16 + sid
    lanes = lax.iota(jnp.int32, 16)
    zpad = jnp.zeros((16,), jnp.float32)
    sent16 = jnp.full((16,), NPAD - 1, jnp.int32)

    pltpu.sync_copy(cs_hbm.at[0, pl.ds(0, 32)], cs_v)

    minus1 = jnp.full((16,), -1, jnp.int32)

    def _init_map(i, c):
        map_v[pl.ds(pl.multiple_of(i * 16, 16), 16)] = minus1
        return c

    lax.fori_loop(0, NPAD // 16, _init_map, 0)

    def _scat_map(j, c):
        pltpu.sync_copy(tgt_hbm.at[pl.ds(pl.multiple_of(j * 64, 64), 64)],
                        tch_v)
        for g in range(4):
            t16 = tch_v[pl.ds(g * 16, 16)]
            plsc.store_scatter(map_v, [t16], lanes + j * 64 + g * 16)
        return c

    lax.fori_loop(0, B // 64, _scat_map, 0)

    def _mrow(j, c):
        jb = pl.multiple_of(wid * (B // NW) + j * 64, 64)
        pltpu.sync_copy(tgt_hbm.at[pl.ds(jb, 64)], tch_v)
        for g in range(4):
            t16 = tch_v[pl.ds(g * 16, 16)]
            mch_v[pl.ds(g * 16, 16)] = plsc.load_gather(map_v, [t16])
        pltpu.sync_copy(mch_v, mrow_hbm.at[pl.ds(jb, 64)])
        return c

    lax.fori_loop(0, B // NW // 64, _mrow, 0)

    def _zero_acc(i, c):
        acc_v[pl.ds(i * 16, 16)] = zpad
        return c

    lax.fori_loop(0, SPT * ACCW // 16, _zero_acc, 0)

    nb_hbm = (nb0_hbm, nb1_hbm)
    cur_hbm = (cur0_hbm, cur1_hbm)
    sid16 = jnp.zeros((16,), jnp.int32) + sid

    def _flush(b, cc):
        fb = pl.multiple_of(b * FB, FB)
        cp1 = pltpu.async_copy(st_hbm.at[pnb_v.at[pl.ds(fb, FB)]], rs_v, sem)
        cp2 = pltpu.async_copy(st_hbm.at[pcur_v.at[pl.ds(fb, FB)]], rd_v,
                               sem2)
        cp1.wait()
        cp2.wait()

        def _grp(g, gc):
            go = pl.multiple_of(g * 16, 16)
            cur16 = pcur_v[pl.ds(fb + go, 16)]
            m16 = plsc.load_gather(map_v, [cur16])
            own = jnp.right_shift(m16, 8) == sid16
            slot16 = jnp.bitwise_and(m16, 255)
            kvec = lanes + go
            ees = []
            for h in range(H):
                col_ss = jnp.zeros((16,), jnp.int32) + (cid * 4 + D + h)
                col_sd = jnp.zeros((16,), jnp.int32) + (cid * 4 + D + 8 + h)
                ss = plsc.load_gather(rs_v, [kvec, col_ss])
                sd = plsc.load_gather(rd_v, [kvec, col_sd])
                csp = plsc.load_gather(
                    cs_v, [jnp.zeros((16,), jnp.int32) + (cid * 4 + 16 + h)])
                ee = jnp.exp(_lrelu(sd + ss) - csp)
                ees.append(jnp.where(own, ee, 0.0))
            for l in range(16):
                k = go + l
                sb2 = slot16[l] * ACCW
                feats = [rs_v[k, pl.ds(j * 16, 16)] for j in range(4)]
                for h in range(H):
                    eh = ees[h][l]
                    for j in range(4):
                        colsl = pl.ds(sb2 + h * D + j * 16, 16)
                        acc_v[colsl] = acc_v[colsl] + eh * feats[j]
                dsl = pl.ds(sb2 + 256, 16)
                evl = [jnp.where(lanes == 0, ees[0][l], 0.0)]
                for h in range(1, H):
                    evl.append(jnp.where(lanes == h, ees[h][l],
                                         evl[h - 1]))
                acc_v[dsl] = acc_v[dsl] + evl[H - 1]
            return gc

        lax.fori_loop(0, FB // 16, _grp, 0)
        return cc

    def _issue(sb, nbdst, curdst, s1):
        for c in range(2):
            @pl.when(cid == c)
            def _cp(c=c):
                pltpu.async_copy(nb_hbm[c].at[pl.ds(sb, STAGE)], nbdst, s1)
                pltpu.async_copy(cur_hbm[c].at[pl.ds(sb, STAGE)], curdst, s1)

    def _await(nbdst, curdst, s1):
        pltpu.make_async_copy(nb0_hbm.at[pl.ds(0, STAGE)], nbdst, s1).wait()
        pltpu.make_async_copy(cur0_hbm.at[pl.ds(0, STAGE)], curdst, s1).wait()

    def _scan_stage(nbuf, cbuf, cnt):
        def _scan(go, cn):
            go = pl.multiple_of(go, 16)
            cur16 = cbuf[pl.ds(go, 16)]
            m16 = plsc.load_gather(map_v, [cur16])
            own = jnp.right_shift(m16, 8) == sid16
            plsc.store_compressed(pnb_v.at[pl.ds(cn, 16)],
                                  nbuf[pl.ds(go, 16)], mask=own)
            plsc.store_compressed(pcur_v.at[pl.ds(cn, 16)], cur16, mask=own)
            npop = plsc.all_reduce_population_count(own)
            return cn + npop[0]

        cnt = plsc.parallel_loop(0, STAGE, 16, unroll=4, carry=cnt)(_scan)
        nfull = cnt // FB
        lax.fori_loop(0, nfull, _flush, 0)
        rb = pl.multiple_of(nfull * FB, FB)
        for g in range(FB // 16):
            tnb = pnb_v[pl.ds(rb + g * 16, 16)]
            tcur = pcur_v[pl.ds(rb + g * 16, 16)]
            pnb_v[pl.ds(g * 16, 16)] = tnb
            pcur_v[pl.ds(g * 16, 16)] = tcur
        return cnt - nfull * FB

    _issue(pl.multiple_of(0, STAGE), nba_v, cura_v, sema)

    def _pair(p, cnt):
        st2 = pl.multiple_of(2 * p * STAGE, STAGE)
        _issue(st2 + STAGE, nbb_v, curb_v, semb)
        _await(nba_v, cura_v, sema)
        cnt = _scan_stage(nba_v, cura_v, cnt)
        nxt = pl.multiple_of((st2 + 2 * STAGE) % (NSTG * STAGE), STAGE)
        _issue(nxt, nba_v, cura_v, sema)
        _await(nbb_v, curb_v, semb)
        cnt = _scan_stage(nbb_v, curb_v, cnt)
        return cnt

    cnt = lax.fori_loop(0, NSTG // 2, _pair, jnp.int32(0))
    _await(nba_v, cura_v, sema)
    for p in range(FB // 16):
        plsc.store_scatter(pnb_v, [cnt + p * 16 + lanes], sent16)
        plsc.store_scatter(pcur_v, [cnt + p * 16 + lanes], sent16)
    lax.fori_loop(0, (cnt + FB - 1) // FB, _flush, 0)

    def _norm(g, c):
        def _row(l, cc):
            kb = (g * 8 + l) * ACCW
            dvec = acc_v[pl.ds(kb + 256, 16)]
            ivec = jnp.where(dvec > 0, 1.0 / dvec, 0.0)
            for h in range(H):
                inv = ivec[h]
                for j in range(4):
                    f = h * D + j * 16
                    drn_v[l, pl.ds(f, 16)] = acc_v[pl.ds(kb + f, 16)] * inv
            return cc

        lax.fori_loop(0, 8, _row, 0)
        r0 = pl.multiple_of(sid * SPT + g * 8, 8)
        for c in range(2):
            @pl.when(cid == c)
            def _wr(c=c):
                dst = hmn0_hbm if c == 0 else hmn1_hbm
                pltpu.sync_copy(drn_v, dst.at[pl.ds(r0, 8)])
        return c

    lax.fori_loop(0, SPT // 8, _norm, 0)


def _edge_call(tgt, nb0, cur0, nb1, cur1, srctab, cs):
    mesh = plsc.VectorSubcoreMesh(core_axis_name="c", subcore_axis_name="s")
    f32 = jnp.float32
    kern = pl.kernel(
        _edge_body,
        out_type=[
            jax.ShapeDtypeStruct((B, 256), f32),
            jax.ShapeDtypeStruct((B, 256), f32),
            jax.ShapeDtypeStruct((B,), jnp.int32),
        ],
        mesh=mesh,
        compiler_params=pltpu.CompilerParams(needs_layout_passes=False),
        scratch_types=[
            pltpu.VMEM((NPAD,), jnp.int32),
            pltpu.VMEM((SPT * ACCW,), f32),
            pltpu.VMEM((PCAP,), jnp.int32),
            pltpu.VMEM((PCAP,), jnp.int32),
            pltpu.VMEM((FB, 128), f32),
            pltpu.VMEM((FB, 128), f32),
            pltpu.VMEM((STAGE,), jnp.int32),
            pltpu.VMEM((STAGE,), jnp.int32),
            pltpu.VMEM((STAGE,), jnp.int32),
            pltpu.VMEM((STAGE,), jnp.int32),
            pltpu.VMEM((64,), jnp.int32),
            pltpu.VMEM((64,), jnp.int32),
            pltpu.VMEM((32,), f32),
            pltpu.VMEM((8, 256), f32),
            pltpu.SemaphoreType.DMA,
            pltpu.SemaphoreType.DMA,
            pltpu.SemaphoreType.DMA,
            pltpu.SemaphoreType.DMA,
        ],
    )
    return kern(tgt, nb0, cur0, nb1, cur1, srctab, cs)


def _gath_body(hmn0_hbm, hmn1_hbm, mrow_hbm, hm0, hm1, mr_v, buf_v, sem):
    cid = lax.axis_index("c")
    sid = lax.axis_index("s")
    wid = cid * 16 + sid
    base = pl.multiple_of(wid * (B // 32), B // 32)
    pltpu.sync_copy(mrow_hbm.at[pl.ds(base, B // 32)], mr_v)
    for m in range(M):
        src = hmn0_hbm if m == 0 else hmn1_hbm
        dst = hm0 if m == 0 else hm1
        pltpu.async_copy(src.at[mr_v], buf_v, sem).wait()
        pltpu.sync_copy(buf_v, dst.at[pl.ds(base, B // 32)])


def _gath_call(hmn0, hmn1, mrow):
    mesh = plsc.VectorSubcoreMesh(core_axis_name="c", subcore_axis_name="s")
    f32 = jnp.float32
    kern = pl.kernel(
        _gath_body,
        out_type=[
            jax.ShapeDtypeStruct((B, 256), f32),
            jax.ShapeDtypeStruct((B, 256), f32),
        ],
        mesh=mesh,
        compiler_params=pltpu.CompilerParams(needs_layout_passes=False),
        scratch_types=[
            pltpu.VMEM((B // 32,), jnp.int32),
            pltpu.VMEM((B // 32, 256), f32),
            pltpu.SemaphoreType.DMA,
        ],
    )
    return kern(hmn0, hmn1, mrow)


def _post_body(h0_ref, h1_ref, ws_ref, bs_ref, as_ref, wc_ref, bc_ref,
               log_ref, emb_ref):
    h0 = _lrelu(h0_ref[...])
    h1 = _lrelu(h1_ref[...])

    def att(h):
        s = jnp.tanh(
            lax.dot_general(h, ws_ref[...], (((1,), (1,)), ((), ())),
                            preferred_element_type=jnp.float32) + bs_ref[...])
        return jnp.mean(jnp.sum(as_ref[...] * s, axis=1))

    a0 = att(h0)
    a1 = att(h1)
    mx = jnp.maximum(a0, a1)
    e0 = jnp.exp(a0 - mx)
    e1 = jnp.exp(a1 - mx)
    b0 = e0 / (e0 + e1)
    b1 = e1 / (e0 + e1)
    emb = b0 * h0 + b1 * h1
    emb_ref[...] = emb
    log_ref[...] = lax.dot_general(emb, wc_ref[...], (((1,), (1,)), ((), ())),
                                   preferred_element_type=jnp.float32) + bc_ref[...]


def _post_call(h0, h1, wsem, bsem, asem, wcls, bcls):
    return pl.pallas_call(
        _post_body,
        out_shape=[
            jax.ShapeDtypeStruct((B, 16), jnp.float32),
            jax.ShapeDtypeStruct((B, H * D), jnp.float32),
        ],
    )(h0, h1, wsem, bsem, asem, wcls, bcls)


def kernel(target_nodes, metapath_list, node_type_mapping, node_feature_list,
           W_proj, b_proj, attn, W_sem, b_sem, a_sem, W_cls, b_cls):
    f32 = jnp.float32
    X = node_feature_list[0]
    Xp = jnp.pad(X, ((0, NPAD - N), (0, 0)))
    ntm = jnp.pad(node_type_mapping, (0, NPAD - N), constant_values=1)
    maskf = (ntm == 0).astype(f32)[:, None]
    a_r = attn.reshape(M, H, 2 * D)
    AW = jnp.concatenate(
        [a_r[0, :, :D].T, a_r[1, :, :D].T, a_r[0, :, D:].T, a_r[1, :, D:].T],
        axis=1)
    Wt = W_proj.T
    bp = b_proj[None, :]

    nb0 = metapath_list[0, :, 0]
    cur0 = metapath_list[0, :, 1]
    nb1 = metapath_list[1, :, 0]
    cur1 = metapath_list[1, :, 1]

    srctab, cs = _pre_call(Xp, Wt, AW, bp, maskf)

    hmn0, hmn1, mrow = _edge_call(target_nodes, nb0, cur0, nb1, cur1,
                                  srctab, cs)

    hm0, hm1 = _gath_call(hmn0, hmn1, mrow)

    logits, emb = _post_call(hm0, hm1, W_sem, b_sem[None, :], a_sem,
                             W_cls, b_cls[None, :])
    return (logits, emb)


# speedup vs baseline: 82.4652x; 1.0355x over previous
import jax
import jax.numpy as jnp
from jax import lax
from jax.experimental import pallas as pl
from jax.experimental.pallas import tpu as pltpu
from jax.experimental.pallas import tpu_sc as plsc

N = 10000
NPAD = 10240
DRAW = 128
D = 64
H = 4
M = 2
B = 4096
E = 160000
SLOPE = 0.01
NW = 32
STAGE = 1600
NSTG = E // STAGE
FB = 128
PCAP = STAGE + 2 * FB
SPT = B // 16
STW = 48
ACCW = 272
RB = 1024


def _lrelu(x):
    return jnp.where(x > 0, x, SLOPE * x)


def _pre_body(x_ref, wt_ref, aw_ref, bp_ref, mk_ref, st_ref, cs_ref):
    i = pl.program_id(0)
    nfb = (jnp.dot(x_ref[...], wt_ref[...], preferred_element_type=jnp.float32)
           + bp_ref[...]) * mk_ref[...]
    sc = jnp.dot(nfb, aw_ref[...], preferred_element_type=jnp.float32)
    st_ref[...] = jnp.concatenate(
        [nfb, sc[:, 8:16], sc[:, 0:8],
         jnp.zeros((RB, 128 - D - 16), jnp.float32)], axis=1)
    bm = jnp.max(sc, axis=0)

    @pl.when(i == 0)
    def _init():
        cs_ref[...] = jnp.full((1, 128), -1e30, jnp.float32)

    cs_ref[0:1, 0:16] = jnp.maximum(cs_ref[0:1, 0:16], bm[None, :])

    @pl.when(i == pl.num_programs(0) - 1)
    def _fin():
        v = cs_ref[0:1, 0:16]
        cs_ref[0:1, 16:24] = _lrelu(v[:, 0:8] + v[:, 8:16])


def _pre_call(xp, wt, aw, bp, maskf):
    ng = NPAD // RB
    return pl.pallas_call(
        _pre_body,
        grid=(ng,),
        in_specs=[
            pl.BlockSpec((RB, DRAW), lambda i: (i, 0)),
            pl.BlockSpec((DRAW, D), lambda i: (0, 0)),
            pl.BlockSpec((D, 16), lambda i: (0, 0)),
            pl.BlockSpec((1, D), lambda i: (0, 0)),
            pl.BlockSpec((RB, 1), lambda i: (i, 0)),
        ],
        out_specs=[
            pl.BlockSpec((RB, 128), lambda i: (i, 0)),
            pl.BlockSpec((1, 128), lambda i: (0, 0)),
        ],
        out_shape=[
            jax.ShapeDtypeStruct((NPAD, 128), jnp.float32),
            jax.ShapeDtypeStruct((1, 128), jnp.float32),
        ],
    )(xp, wt, aw, bp, maskf)


def _edge_body(tgt_hbm, nb0_hbm, cur0_hbm, nb1_hbm, cur1_hbm, st_hbm, cs_hbm,
               hmn0_hbm, hmn1_hbm, mrow_hbm,
               map_v, acc_v, pnb_v, pcur_v, rs_v, rd_v,
               nba_v, cura_v, nbb_v, curb_v, tch_v, mch_v, cs_v, drn_v,
               sem, sem2, sema, semb):
    cid = lax.axis_index("c")
    sid = lax.axis_index("s")
    wid = cid * 16 + sid
    lanes = lax.iota(jnp.int32, 16)
    zpad = jnp.zeros((16,), jnp.float32)
    sent16 = jnp.full((16,), NPAD - 1, jnp.int32)

    pltpu.sync_copy(cs_hbm.at[0, pl.ds(0, 32)], cs_v)

    minus1 = jnp.full((16,), -1, jnp.int32)

    def _init_map(i):
        map_v[pl.ds(pl.multiple_of(i, 16), 16)] = minus1

    plsc.parallel_loop(0, NPAD, 16, unroll=8)(_init_map)

    def _scat_map(j, c):
        pltpu.sync_copy(tgt_hbm.at[pl.ds(pl.multiple_of(j * 64, 64), 64)],
                        tch_v)
        for g in range(4):
            t16 = tch_v[pl.ds(g * 16, 16)]
            plsc.store_scatter(map_v, [t16], lanes + j * 64 + g * 16)
        return c

    lax.fori_loop(0, B // 64, _scat_map, 0)

    def _mrow(j, c):
        jb = pl.multiple_of(wid * (B // NW) + j * 64, 64)
        pltpu.sync_copy(tgt_hbm.at[pl.ds(jb, 64)], tch_v)
        for g in range(4):
            t16 = tch_v[pl.ds(g * 16, 16)]
            mch_v[pl.ds(g * 16, 16)] = plsc.load_gather(map_v, [t16])
        pltpu.sync_copy(mch_v, mrow_hbm.at[pl.ds(jb, 64)])
        return c

    lax.fori_loop(0, B // NW // 64, _mrow, 0)

    def _zero_acc(i):
        acc_v[pl.ds(pl.multiple_of(i, 16), 16)] = zpad

    plsc.parallel_loop(0, SPT * ACCW, 16, unroll=8)(_zero_acc)

    nb_hbm = (nb0_hbm, nb1_hbm)
    cur_hbm = (cur0_hbm, cur1_hbm)
    sid16 = jnp.zeros((16,), jnp.int32) + sid

    def _flush(b, cc):
        fb = pl.multiple_of(b * FB, FB)
        cp1 = pltpu.async_copy(st_hbm.at[pnb_v.at[pl.ds(fb, FB)]], rs_v, sem)
        cp2 = pltpu.async_copy(st_hbm.at[pcur_v.at[pl.ds(fb, FB)]], rd_v,
                               sem2)
        cp1.wait()
        cp2.wait()

        def _grp(g, gc):
            go = pl.multiple_of(g * 16, 16)
            cur16 = pcur_v[pl.ds(fb + go, 16)]
            m16 = plsc.load_gather(map_v, [cur16])
            own = jnp.right_shift(m16, 8) == sid16
            slot16 = jnp.bitwise_and(m16, 255)
            kvec = lanes + go
            ees = []
            for h in range(H):
                col_ss = jnp.zeros((16,), jnp.int32) + (cid * 4 + D + h)
                col_sd = jnp.zeros((16,), jnp.int32) + (cid * 4 + D + 8 + h)
                ss = plsc.load_gather(rs_v, [kvec, col_ss])
                sd = plsc.load_gather(rd_v, [kvec, col_sd])
                csp = plsc.load_gather(
                    cs_v, [jnp.zeros((16,), jnp.int32) + (cid * 4 + 16 + h)])
                ee = jnp.exp(_lrelu(sd + ss) - csp)
                ees.append(jnp.where(own, ee, 0.0))
            for l in range(16):
                k = go + l
                sb2 = slot16[l] * ACCW
                feats = [rs_v[k, pl.ds(j * 16, 16)] for j in range(4)]
                for h in range(H):
                    eh = ees[h][l]
                    for j in range(4):
                        colsl = pl.ds(sb2 + h * D + j * 16, 16)
                        acc_v[colsl] = acc_v[colsl] + eh * feats[j]
                dsl = pl.ds(sb2 + 256, 16)
                evl = [jnp.where(lanes == 0, ees[0][l], 0.0)]
                for h in range(1, H):
                    evl.append(jnp.where(lanes == h, ees[h][l],
                                         evl[h - 1]))
                acc_v[dsl] = acc_v[dsl] + evl[H - 1]
            return gc

        lax.fori_loop(0, FB // 16, _grp, 0)
        return cc

    def _issue(sb, nbdst, curdst, s1):
        for c in range(2):
            @pl.when(cid == c)
            def _cp(c=c):
                pltpu.async_copy(nb_hbm[c].at[pl.ds(sb, STAGE)], nbdst, s1)
                pltpu.async_copy(cur_hbm[c].at[pl.ds(sb, STAGE)], curdst, s1)

    def _await(nbdst, curdst, s1):
        pltpu.make_async_copy(nb0_hbm.at[pl.ds(0, STAGE)], nbdst, s1).wait()
        pltpu.make_async_copy(cur0_hbm.at[pl.ds(0, STAGE)], curdst, s1).wait()

    def _scan_stage(nbuf, cbuf, cnt):
        def _scan(go, cn):
            go = pl.multiple_of(go, 16)
            cur16 = cbuf[pl.ds(go, 16)]
            m16 = plsc.load_gather(map_v, [cur16])
            own = jnp.right_shift(m16, 8) == sid16
            plsc.store_compressed(pnb_v.at[pl.ds(cn, 16)],
                                  nbuf[pl.ds(go, 16)], mask=own)
            plsc.store_compressed(pcur_v.at[pl.ds(cn, 16)], cur16, mask=own)
            npop = plsc.all_reduce_population_count(own)
            return cn + npop[0]

        cnt = plsc.parallel_loop(0, STAGE, 16, unroll=8, carry=cnt)(_scan)
        nfull = cnt // FB
        lax.fori_loop(0, nfull, _flush, 0)
        rb = pl.multiple_of(nfull * FB, FB)
        for g in range(FB // 16):
            tnb = pnb_v[pl.ds(rb + g * 16, 16)]
            tcur = pcur_v[pl.ds(rb + g * 16, 16)]
            pnb_v[pl.ds(g * 16, 16)] = tnb
            pcur_v[pl.ds(g * 16, 16)] = tcur
        return cnt - nfull * FB

    _issue(pl.multiple_of(0, STAGE), nba_v, cura_v, sema)

    def _pair(p, cnt):
        st2 = pl.multiple_of(2 * p * STAGE, STAGE)
        _issue(st2 + STAGE, nbb_v, curb_v, semb)
        _await(nba_v, cura_v, sema)
        cnt = _scan_stage(nba_v, cura_v, cnt)
        nxt = pl.multiple_of((st2 + 2 * STAGE) % (NSTG * STAGE), STAGE)
        _issue(nxt, nba_v, cura_v, sema)
        _await(nbb_v, curb_v, semb)
        cnt = _scan_stage(nbb_v, curb_v, cnt)
        return cnt

    cnt = lax.fori_loop(0, NSTG // 2, _pair, jnp.int32(0))
    _await(nba_v, cura_v, sema)
    for p in range(FB // 16):
        plsc.store_scatter(pnb_v, [cnt + p * 16 + lanes], sent16)
        plsc.store_scatter(pcur_v, [cnt + p * 16 + lanes], sent16)
    lax.fori_loop(0, (cnt + FB - 1) // FB, _flush, 0)

    def _norm(g, c):
        def _row(l, cc):
            kb = (g * 8 + l) * ACCW
            dvec = acc_v[pl.ds(kb + 256, 16)]
            ivec = jnp.where(dvec > 0, 1.0 / dvec, 0.0)
            for h in range(H):
                inv = ivec[h]
                for j in range(4):
                    f = h * D + j * 16
                    drn_v[l, pl.ds(f, 16)] = acc_v[pl.ds(kb + f, 16)] * inv
            return cc

        lax.fori_loop(0, 8, _row, 0)
        r0 = pl.multiple_of(sid * SPT + g * 8, 8)
        for c in range(2):
            @pl.when(cid == c)
            def _wr(c=c):
                dst = hmn0_hbm if c == 0 else hmn1_hbm
                pltpu.sync_copy(drn_v, dst.at[pl.ds(r0, 8)])
        return c

    lax.fori_loop(0, SPT // 8, _norm, 0)


def _edge_call(tgt, nb0, cur0, nb1, cur1, srctab, cs):
    mesh = plsc.VectorSubcoreMesh(core_axis_name="c", subcore_axis_name="s")
    f32 = jnp.float32
    kern = pl.kernel(
        _edge_body,
        out_type=[
            jax.ShapeDtypeStruct((B, 256), f32),
            jax.ShapeDtypeStruct((B, 256), f32),
            jax.ShapeDtypeStruct((B,), jnp.int32),
        ],
        mesh=mesh,
        compiler_params=pltpu.CompilerParams(needs_layout_passes=False),
        scratch_types=[
            pltpu.VMEM((NPAD,), jnp.int32),
            pltpu.VMEM((SPT * ACCW,), f32),
            pltpu.VMEM((PCAP,), jnp.int32),
            pltpu.VMEM((PCAP,), jnp.int32),
            pltpu.VMEM((FB, 128), f32),
            pltpu.VMEM((FB, 128), f32),
            pltpu.VMEM((STAGE,), jnp.int32),
            pltpu.VMEM((STAGE,), jnp.int32),
            pltpu.VMEM((STAGE,), jnp.int32),
            pltpu.VMEM((STAGE,), jnp.int32),
            pltpu.VMEM((64,), jnp.int32),
            pltpu.VMEM((64,), jnp.int32),
            pltpu.VMEM((32,), f32),
            pltpu.VMEM((8, 256), f32),
            pltpu.SemaphoreType.DMA,
            pltpu.SemaphoreType.DMA,
            pltpu.SemaphoreType.DMA,
            pltpu.SemaphoreType.DMA,
        ],
    )
    return kern(tgt, nb0, cur0, nb1, cur1, srctab, cs)


def _gath_body(hmn0_hbm, hmn1_hbm, mrow_hbm, hm0, hm1, mr_v, buf_v, sem):
    cid = lax.axis_index("c")
    sid = lax.axis_index("s")
    wid = cid * 16 + sid
    base = pl.multiple_of(wid * (B // 32), B // 32)
    pltpu.sync_copy(mrow_hbm.at[pl.ds(base, B // 32)], mr_v)
    for m in range(M):
        src = hmn0_hbm if m == 0 else hmn1_hbm
        dst = hm0 if m == 0 else hm1
        pltpu.async_copy(src.at[mr_v], buf_v, sem).wait()
        pltpu.sync_copy(buf_v, dst.at[pl.ds(base, B // 32)])


def _gath_call(hmn0, hmn1, mrow):
    mesh = plsc.VectorSubcoreMesh(core_axis_name="c", subcore_axis_name="s")
    f32 = jnp.float32
    kern = pl.kernel(
        _gath_body,
        out_type=[
            jax.ShapeDtypeStruct((B, 256), f32),
            jax.ShapeDtypeStruct((B, 256), f32),
        ],
        mesh=mesh,
        compiler_params=pltpu.CompilerParams(needs_layout_passes=False),
        scratch_types=[
            pltpu.VMEM((B // 32,), jnp.int32),
            pltpu.VMEM((B // 32, 256), f32),
            pltpu.SemaphoreType.DMA,
        ],
    )
    return kern(hmn0, hmn1, mrow)


def _post_body(h0_ref, h1_ref, ws_ref, bs_ref, as_ref, wc_ref, bc_ref,
               log_ref, emb_ref):
    h0 = _lrelu(h0_ref[...])
    h1 = _lrelu(h1_ref[...])

    def att(h):
        s = jnp.tanh(
            lax.dot_general(h, ws_ref[...], (((1,), (1,)), ((), ())),
                            preferred_element_type=jnp.float32) + bs_ref[...])
        return jnp.mean(jnp.sum(as_ref[...] * s, axis=1))

    a0 = att(h0)
    a1 = att(h1)
    mx = jnp.maximum(a0, a1)
    e0 = jnp.exp(a0 - mx)
    e1 = jnp.exp(a1 - mx)
    b0 = e0 / (e0 + e1)
    b1 = e1 / (e0 + e1)
    emb = b0 * h0 + b1 * h1
    emb_ref[...] = emb
    log_ref[...] = lax.dot_general(emb, wc_ref[...], (((1,), (1,)), ((), ())),
                                   preferred_element_type=jnp.float32) + bc_ref[...]


def _post_call(h0, h1, wsem, bsem, asem, wcls, bcls):
    return pl.pallas_call(
        _post_body,
        out_shape=[
            jax.ShapeDtypeStruct((B, 16), jnp.float32),
            jax.ShapeDtypeStruct((B, H * D), jnp.float32),
        ],
    )(h0, h1, wsem, bsem, asem, wcls, bcls)


def kernel(target_nodes, metapath_list, node_type_mapping, node_feature_list,
           W_proj, b_proj, attn, W_sem, b_sem, a_sem, W_cls, b_cls):
    f32 = jnp.float32
    X = node_feature_list[0]
    Xp = jnp.pad(X, ((0, NPAD - N), (0, 0)))
    ntm = jnp.pad(node_type_mapping, (0, NPAD - N), constant_values=1)
    maskf = (ntm == 0).astype(f32)[:, None]
    a_r = attn.reshape(M, H, 2 * D)
    AW = jnp.concatenate(
        [a_r[0, :, :D].T, a_r[1, :, :D].T, a_r[0, :, D:].T, a_r[1, :, D:].T],
        axis=1)
    Wt = W_proj.T
    bp = b_proj[None, :]

    nb0 = metapath_list[0, :, 0]
    cur0 = metapath_list[0, :, 1]
    nb1 = metapath_list[1, :, 0]
    cur1 = metapath_list[1, :, 1]

    srctab, cs = _pre_call(Xp, Wt, AW, bp, maskf)

    hmn0, hmn1, mrow = _edge_call(target_nodes, nb0, cur0, nb1, cur1,
                                  srctab, cs)

    hm0, hm1 = _gath_call(hmn0, hmn1, mrow)

    logits, emb = _post_call(hm0, hm1, W_sem, b_sem[None, :], a_sem,
                             W_cls, b_cls[None, :])
    return (logits, emb)


# speedup vs baseline: 82.4664x; 1.0000x over previous
import jax
import jax.numpy as jnp
from jax import lax
from jax.experimental import pallas as pl
from jax.experimental.pallas import tpu as pltpu
from jax.experimental.pallas import tpu_sc as plsc

N = 10000
NPAD = 10240
DRAW = 128
D = 64
H = 4
M = 2
B = 4096
E = 160000
SLOPE = 0.01
NW = 32
STAGE = 1600
NSTG = E // STAGE
FB = 128
PCAP = STAGE + 2 * FB
SPT = B // 16
ACCW = 272
RB = 1024


def _lrelu(x):
    return jnp.where(x > 0, x, SLOPE * x)


def _pre_body(x_ref, wt_ref, aw_ref, bp_ref, mk_ref, st_ref, cs_ref):
    i = pl.program_id(0)
    nfb = (jnp.dot(x_ref[...], wt_ref[...], preferred_element_type=jnp.float32)
           + bp_ref[...]) * mk_ref[...]
    sc = jnp.dot(nfb, aw_ref[...], preferred_element_type=jnp.float32)
    st_ref[...] = jnp.concatenate(
        [nfb, sc[:, 8:16], sc[:, 0:8],
         jnp.zeros((RB, 128 - D - 16), jnp.float32)], axis=1)
    bm = jnp.max(sc, axis=0)

    @pl.when(i == 0)
    def _init():
        cs_ref[...] = jnp.full((1, 128), -1e30, jnp.float32)

    cs_ref[0:1, 0:16] = jnp.maximum(cs_ref[0:1, 0:16], bm[None, :])

    @pl.when(i == pl.num_programs(0) - 1)
    def _fin():
        v = cs_ref[0:1, 0:16]
        cs_ref[0:1, 16:24] = _lrelu(v[:, 0:8] + v[:, 8:16])


def _pre_call(xp, wt, aw, bp, maskf):
    ng = NPAD // RB
    return pl.pallas_call(
        _pre_body,
        grid=(ng,),
        in_specs=[
            pl.BlockSpec((RB, DRAW), lambda i: (i, 0)),
            pl.BlockSpec((DRAW, D), lambda i: (0, 0)),
            pl.BlockSpec((D, 16), lambda i: (0, 0)),
            pl.BlockSpec((1, D), lambda i: (0, 0)),
            pl.BlockSpec((RB, 1), lambda i: (i, 0)),
        ],
        out_specs=[
            pl.BlockSpec((RB, 128), lambda i: (i, 0)),
            pl.BlockSpec((1, 128), lambda i: (0, 0)),
        ],
        out_shape=[
            jax.ShapeDtypeStruct((NPAD, 128), jnp.float32),
            jax.ShapeDtypeStruct((1, 128), jnp.float32),
        ],
    )(xp, wt, aw, bp, maskf)


def _edge_body(tgt_hbm, nb0_hbm, cur0_hbm, nb1_hbm, cur1_hbm, st_hbm, cs_hbm,
               hmn0_hbm, hmn1_hbm, mrow_hbm,
               map_v, acc_v, pnb_v, pcur_v, rs_v, rd_v,
               nba_v, cura_v, nbb_v, curb_v, tch_v, mch_v, cs_v, drn_v,
               sem, sem2, sema, semb):
    cid = lax.axis_index("c")
    sid = lax.axis_index("s")
    wid = cid * 16 + sid
    lanes = lax.iota(jnp.int32, 16)
    zpad = jnp.zeros((16,), jnp.float32)
    sent16 = jnp.full((16,), NPAD - 1, jnp.int32)

    pltpu.sync_copy(cs_hbm.at[0, pl.ds(0, 32)], cs_v)

    minus1 = jnp.full((16,), -1, jnp.int32)

    def _init_map(i):
        map_v[pl.ds(pl.multiple_of(i, 16), 16)] = minus1

    plsc.parallel_loop(0, NPAD, 16, unroll=8)(_init_map)

    def _scat_map(j, c):
        pltpu.sync_copy(tgt_hbm.at[pl.ds(pl.multiple_of(j * 64, 64), 64)],
                        tch_v)
        for g in range(4):
            t16 = tch_v[pl.ds(g * 16, 16)]
            plsc.store_scatter(map_v, [t16], lanes + j * 64 + g * 16)
        return c

    lax.fori_loop(0, B // 64, _scat_map, 0)

    def _mrow(j, c):
        jb = pl.multiple_of(wid * (B // NW) + j * 64, 64)
        pltpu.sync_copy(tgt_hbm.at[pl.ds(jb, 64)], tch_v)
        for g in range(4):
            t16 = tch_v[pl.ds(g * 16, 16)]
            mch_v[pl.ds(g * 16, 16)] = plsc.load_gather(map_v, [t16])
        pltpu.sync_copy(mch_v, mrow_hbm.at[pl.ds(jb, 64)])
        return c

    lax.fori_loop(0, B // NW // 64, _mrow, 0)

    def _zero_acc(i):
        acc_v[pl.ds(pl.multiple_of(i, 16), 16)] = zpad

    plsc.parallel_loop(0, SPT * ACCW, 16, unroll=8)(_zero_acc)

    nb_hbm = (nb0_hbm, nb1_hbm)
    cur_hbm = (cur0_hbm, cur1_hbm)
    sid16 = jnp.zeros((16,), jnp.int32) + sid

    def _flush(b, cc):
        fb = pl.multiple_of(b * FB, FB)
        cp1 = pltpu.async_copy(st_hbm.at[pnb_v.at[pl.ds(fb, FB)]], rs_v, sem)
        cp2 = pltpu.async_copy(st_hbm.at[pcur_v.at[pl.ds(fb, FB)]], rd_v,
                               sem2)
        cp1.wait()
        cp2.wait()

        def _grp(g, gc):
            go = pl.multiple_of(g * 16, 16)
            cur16 = pcur_v[pl.ds(fb + go, 16)]
            m16 = plsc.load_gather(map_v, [cur16])
            own = jnp.right_shift(m16, 8) == sid16
            slot16 = jnp.bitwise_and(m16, 255)
            kvec = lanes + go
            ees = []
            for h in range(H):
                col_ss = jnp.zeros((16,), jnp.int32) + (cid * 4 + D + h)
                col_sd = jnp.zeros((16,), jnp.int32) + (cid * 4 + D + 8 + h)
                ss = plsc.load_gather(rs_v, [kvec, col_ss])
                sd = plsc.load_gather(rd_v, [kvec, col_sd])
                csp = plsc.load_gather(
                    cs_v, [jnp.zeros((16,), jnp.int32) + (cid * 4 + 16 + h)])
                ee = jnp.exp(_lrelu(sd + ss) - csp)
                ees.append(jnp.where(own, ee, 0.0))
            for l in range(16):
                k = go + l
                sb2 = slot16[l] * ACCW
                feats = [rs_v[k, pl.ds(j * 16, 16)] for j in range(4)]
                for h in range(H):
                    eh = ees[h][l]
                    for j in range(4):
                        colsl = pl.ds(sb2 + h * D + j * 16, 16)
                        acc_v[colsl] = acc_v[colsl] + eh * feats[j]
                dsl = pl.ds(sb2 + 256, 16)
                evl = [jnp.where(lanes == 0, ees[0][l], 0.0)]
                for h in range(1, H):
                    evl.append(jnp.where(lanes == h, ees[h][l],
                                         evl[h - 1]))
                acc_v[dsl] = acc_v[dsl] + evl[H - 1]
            return gc

        lax.fori_loop(0, FB // 16, _grp, 0)
        return cc

    def _issue(sb, nbdst, curdst, s1):
        for c in range(2):
            @pl.when(cid == c)
            def _cp(c=c):
                pltpu.async_copy(nb_hbm[c].at[pl.ds(sb, STAGE)], nbdst, s1)
                pltpu.async_copy(cur_hbm[c].at[pl.ds(sb, STAGE)], curdst, s1)

    def _await(nbdst, curdst, s1):
        pltpu.make_async_copy(nb0_hbm.at[pl.ds(0, STAGE)], nbdst, s1).wait()
        pltpu.make_async_copy(cur0_hbm.at[pl.ds(0, STAGE)], curdst, s1).wait()

    def _scan_stage(nbuf, cbuf, cnt):
        def _scan(go, cn):
            go = pl.multiple_of(go, 16)
            cur16 = cbuf[pl.ds(go, 16)]
            m16 = plsc.load_gather(map_v, [cur16])
            own = jnp.right_shift(m16, 8) == sid16
            plsc.store_compressed(pnb_v.at[pl.ds(cn, 16)],
                                  nbuf[pl.ds(go, 16)], mask=own)
            plsc.store_compressed(pcur_v.at[pl.ds(cn, 16)], cur16, mask=own)
            npop = plsc.all_reduce_population_count(own)
            return cn + npop[0]

        cnt = plsc.parallel_loop(0, STAGE, 16, unroll=8, carry=cnt)(_scan)
        nfull = cnt // FB
        lax.fori_loop(0, nfull, _flush, 0)
        rb = pl.multiple_of(nfull * FB, FB)
        for g in range(FB // 16):
            tnb = pnb_v[pl.ds(rb + g * 16, 16)]
            tcur = pcur_v[pl.ds(rb + g * 16, 16)]
            pnb_v[pl.ds(g * 16, 16)] = tnb
            pcur_v[pl.ds(g * 16, 16)] = tcur
        return cnt - nfull * FB

    _issue(pl.multiple_of(0, STAGE), nba_v, cura_v, sema)

    def _pair(p, cnt):
        st2 = pl.multiple_of(2 * p * STAGE, STAGE)
        _issue(st2 + STAGE, nbb_v, curb_v, semb)
        _await(nba_v, cura_v, sema)
        cnt = _scan_stage(nba_v, cura_v, cnt)
        nxt = pl.multiple_of((st2 + 2 * STAGE) % (NSTG * STAGE), STAGE)
        _issue(nxt, nba_v, cura_v, sema)
        _await(nbb_v, curb_v, semb)
        cnt = _scan_stage(nbb_v, curb_v, cnt)
        return cnt

    cnt = lax.fori_loop(0, NSTG // 2, _pair, jnp.int32(0))
    _await(nba_v, cura_v, sema)
    for p in range(FB // 16):
        plsc.store_scatter(pnb_v, [cnt + p * 16 + lanes], sent16)
        plsc.store_scatter(pcur_v, [cnt + p * 16 + lanes], sent16)
    lax.fori_loop(0, (cnt + FB - 1) // FB, _flush, 0)

    def _norm(g, c):
        def _row(l, cc):
            kb = (g * 8 + l) * ACCW
            dvec = acc_v[pl.ds(kb + 256, 16)]
            ivec = jnp.where(dvec > 0, 1.0 / dvec, 0.0)
            for h in range(H):
                inv = ivec[h]
                for j in range(4):
                    f = h * D + j * 16
                    drn_v[l, pl.ds(f, 16)] = acc_v[pl.ds(kb + f, 16)] * inv
            return cc

        lax.fori_loop(0, 8, _row, 0)
        r0 = pl.multiple_of(sid * SPT + g * 8, 8)
        for c in range(2):
            @pl.when(cid == c)
            def _wr(c=c):
                dst = hmn0_hbm if c == 0 else hmn1_hbm
                pltpu.sync_copy(drn_v, dst.at[pl.ds(r0, 8)])
        return c

    lax.fori_loop(0, SPT // 8, _norm, 0)


def _edge_call(tgt, nb0, cur0, nb1, cur1, srctab, cs):
    mesh = plsc.VectorSubcoreMesh(core_axis_name="c", subcore_axis_name="s")
    f32 = jnp.float32
    kern = pl.kernel(
        _edge_body,
        out_type=[
            jax.ShapeDtypeStruct((B, 256), f32),
            jax.ShapeDtypeStruct((B, 256), f32),
            jax.ShapeDtypeStruct((B,), jnp.int32),
        ],
        mesh=mesh,
        compiler_params=pltpu.CompilerParams(needs_layout_passes=False),
        scratch_types=[
            pltpu.VMEM((NPAD,), jnp.int32),
            pltpu.VMEM((SPT * ACCW,), f32),
            pltpu.VMEM((PCAP,), jnp.int32),
            pltpu.VMEM((PCAP,), jnp.int32),
            pltpu.VMEM((FB, 128), f32),
            pltpu.VMEM((FB, 128), f32),
            pltpu.VMEM((STAGE,), jnp.int32),
            pltpu.VMEM((STAGE,), jnp.int32),
            pltpu.VMEM((STAGE,), jnp.int32),
            pltpu.VMEM((STAGE,), jnp.int32),
            pltpu.VMEM((64,), jnp.int32),
            pltpu.VMEM((64,), jnp.int32),
            pltpu.VMEM((32,), f32),
            pltpu.VMEM((8, 256), f32),
            pltpu.SemaphoreType.DMA,
            pltpu.SemaphoreType.DMA,
            pltpu.SemaphoreType.DMA,
            pltpu.SemaphoreType.DMA,
        ],
    )
    return kern(tgt, nb0, cur0, nb1, cur1, srctab, cs)


def _gath_body(hmn0_hbm, hmn1_hbm, mrow_hbm, hm0, hm1, mr_v, buf_v, sem):
    cid = lax.axis_index("c")
    sid = lax.axis_index("s")
    wid = cid * 16 + sid
    base = pl.multiple_of(wid * (B // 32), B // 32)
    pltpu.sync_copy(mrow_hbm.at[pl.ds(base, B // 32)], mr_v)
    for m in range(M):
        src = hmn0_hbm if m == 0 else hmn1_hbm
        dst = hm0 if m == 0 else hm1
        pltpu.async_copy(src.at[mr_v], buf_v, sem).wait()
        pltpu.sync_copy(buf_v, dst.at[pl.ds(base, B // 32)])


def _gath_call(hmn0, hmn1, mrow):
    mesh = plsc.VectorSubcoreMesh(core_axis_name="c", subcore_axis_name="s")
    f32 = jnp.float32
    kern = pl.kernel(
        _gath_body,
        out_type=[
            jax.ShapeDtypeStruct((B, 256), f32),
            jax.ShapeDtypeStruct((B, 256), f32),
        ],
        mesh=mesh,
        compiler_params=pltpu.CompilerParams(needs_layout_passes=False),
        scratch_types=[
            pltpu.VMEM((B // 32,), jnp.int32),
            pltpu.VMEM((B // 32, 256), f32),
            pltpu.SemaphoreType.DMA,
        ],
    )
    return kern(hmn0, hmn1, mrow)


def _post_body(h0_ref, h1_ref, ws_ref, bs_ref, as_ref, wc_ref, bc_ref,
               log_ref, emb_ref):
    h0 = _lrelu(h0_ref[...])
    h1 = _lrelu(h1_ref[...])

    def att(h):
        s = jnp.tanh(
            lax.dot_general(h, ws_ref[...], (((1,), (1,)), ((), ())),
                            preferred_element_type=jnp.float32) + bs_ref[...])
        return jnp.mean(jnp.sum(as_ref[...] * s, axis=1))

    a0 = att(h0)
    a1 = att(h1)
    mx = jnp.maximum(a0, a1)
    e0 = jnp.exp(a0 - mx)
    e1 = jnp.exp(a1 - mx)
    b0 = e0 / (e0 + e1)
    b1 = e1 / (e0 + e1)
    emb = b0 * h0 + b1 * h1
    emb_ref[...] = emb
    log_ref[...] = lax.dot_general(emb, wc_ref[...], (((1,), (1,)), ((), ())),
                                   preferred_element_type=jnp.float32) + bc_ref[...]


def _post_call(h0, h1, wsem, bsem, asem, wcls, bcls):
    return pl.pallas_call(
        _post_body,
        out_shape=[
            jax.ShapeDtypeStruct((B, 16), jnp.float32),
            jax.ShapeDtypeStruct((B, H * D), jnp.float32),
        ],
    )(h0, h1, wsem, bsem, asem, wcls, bcls)


def kernel(target_nodes, metapath_list, node_type_mapping, node_feature_list,
           W_proj, b_proj, attn, W_sem, b_sem, a_sem, W_cls, b_cls):
    f32 = jnp.float32
    X = node_feature_list[0]
    Xp = jnp.pad(X, ((0, NPAD - N), (0, 0)))
    ntm = jnp.pad(node_type_mapping, (0, NPAD - N), constant_values=1)
    maskf = (ntm == 0).astype(f32)[:, None]
    a_r = attn.reshape(M, H, 2 * D)
    AW = jnp.concatenate(
        [a_r[0, :, :D].T, a_r[1, :, :D].T, a_r[0, :, D:].T, a_r[1, :, D:].T],
        axis=1)
    Wt = W_proj.T
    bp = b_proj[None, :]

    nb0 = metapath_list[0, :, 0]
    cur0 = metapath_list[0, :, 1]
    nb1 = metapath_list[1, :, 0]
    cur1 = metapath_list[1, :, 1]

    srctab, cs = _pre_call(Xp, Wt, AW, bp, maskf)

    hmn0, hmn1, mrow = _edge_call(target_nodes, nb0, cur0, nb1, cur1,
                                  srctab, cs)

    hm0, hm1 = _gath_call(hmn0, hmn1, mrow)

    logits, emb = _post_call(hm0, hm1, W_sem, b_sem[None, :], a_sem,
                             W_cls, b_cls[None, :])
    return (logits, emb)


# speedup vs baseline: 90.7000x; 1.0998x over previous
import jax
import jax.numpy as jnp
from jax import lax
from jax.experimental import pallas as pl
from jax.experimental.pallas import tpu as pltpu
from jax.experimental.pallas import tpu_sc as plsc

N = 10000
NPAD = 10240
DRAW = 128
D = 64
H = 4
M = 2
B = 4096
E = 160000
SLOPE = 0.01
NW = 32
STAGE = 1600
NSTG = E // STAGE
FB = 128
PCAP = STAGE + 2 * FB
SPT = B // 16
ACCW = 272
RB = 1024


def _lrelu(x):
    return jnp.where(x > 0, x, SLOPE * x)


def _pre_body(x_ref, wt_ref, aw_ref, bp_ref, mk_ref, st_ref, cs_ref):
    i = pl.program_id(0)
    nfb = (jnp.dot(x_ref[...], wt_ref[...], preferred_element_type=jnp.float32)
           + bp_ref[...]) * mk_ref[...]
    sc = jnp.dot(nfb, aw_ref[...], preferred_element_type=jnp.float32)
    st_ref[...] = jnp.concatenate(
        [nfb, sc[:, 8:16], sc[:, 0:8],
         jnp.zeros((RB, 128 - D - 16), jnp.float32)], axis=1)
    bm = jnp.max(sc, axis=0)

    @pl.when(i == 0)
    def _init():
        cs_ref[...] = jnp.full((1, 128), -1e30, jnp.float32)

    cs_ref[0:1, 0:16] = jnp.maximum(cs_ref[0:1, 0:16], bm[None, :])

    @pl.when(i == pl.num_programs(0) - 1)
    def _fin():
        v = cs_ref[0:1, 0:16]
        cs_ref[0:1, 16:24] = _lrelu(v[:, 0:8] + v[:, 8:16])


def _pre_call(xp, wt, aw, bp, maskf):
    ng = NPAD // RB
    return pl.pallas_call(
        _pre_body,
        grid=(ng,),
        in_specs=[
            pl.BlockSpec((RB, DRAW), lambda i: (i, 0)),
            pl.BlockSpec((DRAW, D), lambda i: (0, 0)),
            pl.BlockSpec((D, 16), lambda i: (0, 0)),
            pl.BlockSpec((1, D), lambda i: (0, 0)),
            pl.BlockSpec((RB, 1), lambda i: (i, 0)),
        ],
        out_specs=[
            pl.BlockSpec((RB, 128), lambda i: (i, 0)),
            pl.BlockSpec((1, 128), lambda i: (0, 0)),
        ],
        out_shape=[
            jax.ShapeDtypeStruct((NPAD, 128), jnp.float32),
            jax.ShapeDtypeStruct((1, 128), jnp.float32),
        ],
    )(xp, wt, aw, bp, maskf)


def _edge_body(tgt_hbm, nb0_hbm, cur0_hbm, nb1_hbm, cur1_hbm, st_hbm, cs_hbm,
               hmn0_hbm, hmn1_hbm, mrow_hbm,
               map_v, acc_v, pnb_v, pcur_v, rs_v, rd_v,
               nba_v, cura_v, nbb_v, curb_v, tch_v, mch_v, cs_v, drn_v,
               sem, sem2, sema, semb):
    cid = lax.axis_index("c")
    sid = lax.axis_index("s")
    wid = cid * 16 + sid
    lanes = lax.iota(jnp.int32, 16)
    zpad = jnp.zeros((16,), jnp.float32)
    sent16 = jnp.full((16,), NPAD - 1, jnp.int32)

    pltpu.sync_copy(cs_hbm.at[0, pl.ds(0, 32)], cs_v)

    minus1 = jnp.full((16,), -1, jnp.int32)

    def _init_map(i):
        map_v[pl.ds(pl.multiple_of(i, 16), 16)] = minus1

    plsc.parallel_loop(0, NPAD, 16, unroll=8)(_init_map)

    def _scat_map(j, c):
        pltpu.sync_copy(tgt_hbm.at[pl.ds(pl.multiple_of(j * 64, 64), 64)],
                        tch_v)
        for g in range(4):
            t16 = tch_v[pl.ds(g * 16, 16)]
            plsc.store_scatter(map_v, [t16], lanes + j * 64 + g * 16)
        return c

    lax.fori_loop(0, B // 64, _scat_map, 0)

    def _mrow(j, c):
        jb = pl.multiple_of(wid * (B // NW) + j * 64, 64)
        pltpu.sync_copy(tgt_hbm.at[pl.ds(jb, 64)], tch_v)
        for g in range(4):
            t16 = tch_v[pl.ds(g * 16, 16)]
            mch_v[pl.ds(g * 16, 16)] = plsc.load_gather(map_v, [t16])
        pltpu.sync_copy(mch_v, mrow_hbm.at[pl.ds(jb, 64)])
        return c

    lax.fori_loop(0, B // NW // 64, _mrow, 0)

    def _zero_acc(i):
        acc_v[pl.ds(pl.multiple_of(i, 16), 16)] = zpad

    plsc.parallel_loop(0, SPT * ACCW, 16, unroll=8)(_zero_acc)

    nb_hbm = (nb0_hbm, nb1_hbm)
    cur_hbm = (cur0_hbm, cur1_hbm)
    sid16 = jnp.zeros((16,), jnp.int32) + sid

    def _flush(b, cc):
        fb = pl.multiple_of(b * FB, FB)
        cp1 = pltpu.async_copy(st_hbm.at[pnb_v.at[pl.ds(fb, FB)]], rs_v, sem)
        cp2 = pltpu.async_copy(st_hbm.at[pcur_v.at[pl.ds(fb, FB)]], rd_v,
                               sem2)
        cp1.wait()
        cp2.wait()

        def _grp(g, gc):
            go = pl.multiple_of(g * 16, 16)
            cur16 = pcur_v[pl.ds(fb + go, 16)]
            m16 = plsc.load_gather(map_v, [cur16])
            own = jnp.right_shift(m16, 8) == sid16
            slot16 = jnp.bitwise_and(m16, 255)
            kvec = lanes + go
            ees = []
            for h in range(H):
                col_ss = jnp.zeros((16,), jnp.int32) + (cid * 4 + D + h)
                col_sd = jnp.zeros((16,), jnp.int32) + (cid * 4 + D + 8 + h)
                ss = plsc.load_gather(rs_v, [kvec, col_ss])
                sd = plsc.load_gather(rd_v, [kvec, col_sd])
                csp = plsc.load_gather(
                    cs_v, [jnp.zeros((16,), jnp.int32) + (cid * 4 + 16 + h)])
                ee = jnp.exp(_lrelu(sd + ss) - csp)
                ees.append(jnp.where(own, ee, 0.0))
            for l in range(16):
                k = go + l
                sb2 = slot16[l] * ACCW
                feats = [rs_v[k, pl.ds(j * 16, 16)] for j in range(4)]
                for h in range(H):
                    eh = ees[h][l]
                    for j in range(4):
                        plsc.addupdate(
                            acc_v.at[pl.ds(sb2 + h * D + j * 16, 16)],
                            eh * feats[j])
                evl = [jnp.where(lanes == 0, ees[0][l], 0.0)]
                for h in range(1, H):
                    evl.append(jnp.where(lanes == h, ees[h][l],
                                         evl[h - 1]))
                plsc.addupdate(acc_v.at[pl.ds(sb2 + 256, 16)], evl[H - 1])
            return gc

        lax.fori_loop(0, FB // 16, _grp, 0)
        return cc

    def _issue(sb, nbdst, curdst, s1):
        for c in range(2):
            @pl.when(cid == c)
            def _cp(c=c):
                pltpu.async_copy(nb_hbm[c].at[pl.ds(sb, STAGE)], nbdst, s1)
                pltpu.async_copy(cur_hbm[c].at[pl.ds(sb, STAGE)], curdst, s1)

    def _await(nbdst, curdst, s1):
        pltpu.make_async_copy(nb0_hbm.at[pl.ds(0, STAGE)], nbdst, s1).wait()
        pltpu.make_async_copy(cur0_hbm.at[pl.ds(0, STAGE)], curdst, s1).wait()

    def _scan_stage(nbuf, cbuf, cnt):
        def _scan(go, cn):
            go = pl.multiple_of(go, 16)
            cur16 = cbuf[pl.ds(go, 16)]
            m16 = plsc.load_gather(map_v, [cur16])
            own = jnp.right_shift(m16, 8) == sid16
            plsc.store_compressed(pnb_v.at[pl.ds(cn, 16)],
                                  nbuf[pl.ds(go, 16)], mask=own)
            plsc.store_compressed(pcur_v.at[pl.ds(cn, 16)], cur16, mask=own)
            npop = plsc.all_reduce_population_count(own)
            return cn + npop[0]

        cnt = plsc.parallel_loop(0, STAGE, 16, unroll=8, carry=cnt)(_scan)
        nfull = cnt // FB
        lax.fori_loop(0, nfull, _flush, 0)
        rb = pl.multiple_of(nfull * FB, FB)
        for g in range(FB // 16):
            tnb = pnb_v[pl.ds(rb + g * 16, 16)]
            tcur = pcur_v[pl.ds(rb + g * 16, 16)]
            pnb_v[pl.ds(g * 16, 16)] = tnb
            pcur_v[pl.ds(g * 16, 16)] = tcur
        return cnt - nfull * FB

    _issue(pl.multiple_of(0, STAGE), nba_v, cura_v, sema)

    def _pair(p, cnt):
        st2 = pl.multiple_of(2 * p * STAGE, STAGE)
        _issue(st2 + STAGE, nbb_v, curb_v, semb)
        _await(nba_v, cura_v, sema)
        cnt = _scan_stage(nba_v, cura_v, cnt)
        nxt = pl.multiple_of((st2 + 2 * STAGE) % (NSTG * STAGE), STAGE)
        _issue(nxt, nba_v, cura_v, sema)
        _await(nbb_v, curb_v, semb)
        cnt = _scan_stage(nbb_v, curb_v, cnt)
        return cnt

    cnt = lax.fori_loop(0, NSTG // 2, _pair, jnp.int32(0))
    _await(nba_v, cura_v, sema)
    for p in range(FB // 16):
        plsc.store_scatter(pnb_v, [cnt + p * 16 + lanes], sent16)
        plsc.store_scatter(pcur_v, [cnt + p * 16 + lanes], sent16)
    lax.fori_loop(0, (cnt + FB - 1) // FB, _flush, 0)

    def _norm(g, c):
        def _row(l, cc):
            kb = (g * 8 + l) * ACCW
            dvec = acc_v[pl.ds(kb + 256, 16)]
            ivec = jnp.where(dvec > 0, 1.0 / dvec, 0.0)
            for h in range(H):
                inv = ivec[h]
                for j in range(4):
                    f = h * D + j * 16
                    drn_v[l, pl.ds(f, 16)] = acc_v[pl.ds(kb + f, 16)] * inv
            return cc

        lax.fori_loop(0, 8, _row, 0)
        r0 = pl.multiple_of(sid * SPT + g * 8, 8)
        for c in range(2):
            @pl.when(cid == c)
            def _wr(c=c):
                dst = hmn0_hbm if c == 0 else hmn1_hbm
                pltpu.sync_copy(drn_v, dst.at[pl.ds(r0, 8)])
        return c

    lax.fori_loop(0, SPT // 8, _norm, 0)


def _edge_call(tgt, nb0, cur0, nb1, cur1, srctab, cs):
    mesh = plsc.VectorSubcoreMesh(core_axis_name="c", subcore_axis_name="s")
    f32 = jnp.float32
    kern = pl.kernel(
        _edge_body,
        out_type=[
            jax.ShapeDtypeStruct((B, 256), f32),
            jax.ShapeDtypeStruct((B, 256), f32),
            jax.ShapeDtypeStruct((B,), jnp.int32),
        ],
        mesh=mesh,
        compiler_params=pltpu.CompilerParams(needs_layout_passes=False),
        scratch_types=[
            pltpu.VMEM((NPAD,), jnp.int32),
            pltpu.VMEM((SPT * ACCW,), f32),
            pltpu.VMEM((PCAP,), jnp.int32),
            pltpu.VMEM((PCAP,), jnp.int32),
            pltpu.VMEM((FB, 128), f32),
            pltpu.VMEM((FB, 128), f32),
            pltpu.VMEM((STAGE,), jnp.int32),
            pltpu.VMEM((STAGE,), jnp.int32),
            pltpu.VMEM((STAGE,), jnp.int32),
            pltpu.VMEM((STAGE,), jnp.int32),
            pltpu.VMEM((64,), jnp.int32),
            pltpu.VMEM((64,), jnp.int32),
            pltpu.VMEM((32,), f32),
            pltpu.VMEM((8, 256), f32),
            pltpu.SemaphoreType.DMA,
            pltpu.SemaphoreType.DMA,
            pltpu.SemaphoreType.DMA,
            pltpu.SemaphoreType.DMA,
        ],
    )
    return kern(tgt, nb0, cur0, nb1, cur1, srctab, cs)


def _gath_body(hmn0_hbm, hmn1_hbm, mrow_hbm, hm0, hm1, mr_v, buf_v, sem):
    cid = lax.axis_index("c")
    sid = lax.axis_index("s")
    wid = cid * 16 + sid
    base = pl.multiple_of(wid * (B // 32), B // 32)
    pltpu.sync_copy(mrow_hbm.at[pl.ds(base, B // 32)], mr_v)
    for m in range(M):
        src = hmn0_hbm if m == 0 else hmn1_hbm
        dst = hm0 if m == 0 else hm1
        pltpu.async_copy(src.at[mr_v], buf_v, sem).wait()
        pltpu.sync_copy(buf_v, dst.at[pl.ds(base, B // 32)])


def _gath_call(hmn0, hmn1, mrow):
    mesh = plsc.VectorSubcoreMesh(core_axis_name="c", subcore_axis_name="s")
    f32 = jnp.float32
    kern = pl.kernel(
        _gath_body,
        out_type=[
            jax.ShapeDtypeStruct((B, 256), f32),
            jax.ShapeDtypeStruct((B, 256), f32),
        ],
        mesh=mesh,
        compiler_params=pltpu.CompilerParams(needs_layout_passes=False),
        scratch_types=[
            pltpu.VMEM((B // 32,), jnp.int32),
            pltpu.VMEM((B // 32, 256), f32),
            pltpu.SemaphoreType.DMA,
        ],
    )
    return kern(hmn0, hmn1, mrow)


def _post_body(h0_ref, h1_ref, ws_ref, bs_ref, as_ref, wc_ref, bc_ref,
               log_ref, emb_ref):
    h0 = _lrelu(h0_ref[...])
    h1 = _lrelu(h1_ref[...])

    def att(h):
        s = jnp.tanh(
            lax.dot_general(h, ws_ref[...], (((1,), (1,)), ((), ())),
                            preferred_element_type=jnp.float32) + bs_ref[...])
        return jnp.mean(jnp.sum(as_ref[...] * s, axis=1))

    a0 = att(h0)
    a1 = att(h1)
    mx = jnp.maximum(a0, a1)
    e0 = jnp.exp(a0 - mx)
    e1 = jnp.exp(a1 - mx)
    b0 = e0 / (e0 + e1)
    b1 = e1 / (e0 + e1)
    emb = b0 * h0 + b1 * h1
    emb_ref[...] = emb
    log_ref[...] = lax.dot_general(emb, wc_ref[...], (((1,), (1,)), ((), ())),
                                   preferred_element_type=jnp.float32) + bc_ref[...]


def _post_call(h0, h1, wsem, bsem, asem, wcls, bcls):
    return pl.pallas_call(
        _post_body,
        out_shape=[
            jax.ShapeDtypeStruct((B, 16), jnp.float32),
            jax.ShapeDtypeStruct((B, H * D), jnp.float32),
        ],
    )(h0, h1, wsem, bsem, asem, wcls, bcls)


def kernel(target_nodes, metapath_list, node_type_mapping, node_feature_list,
           W_proj, b_proj, attn, W_sem, b_sem, a_sem, W_cls, b_cls):
    f32 = jnp.float32
    X = node_feature_list[0]
    Xp = jnp.pad(X, ((0, NPAD - N), (0, 0)))
    ntm = jnp.pad(node_type_mapping, (0, NPAD - N), constant_values=1)
    maskf = (ntm == 0).astype(f32)[:, None]
    a_r = attn.reshape(M, H, 2 * D)
    AW = jnp.concatenate(
        [a_r[0, :, :D].T, a_r[1, :, :D].T, a_r[0, :, D:].T, a_r[1, :, D:].T],
        axis=1)
    Wt = W_proj.T
    bp = b_proj[None, :]

    nb0 = metapath_list[0, :, 0]
    cur0 = metapath_list[0, :, 1]
    nb1 = metapath_list[1, :, 0]
    cur1 = metapath_list[1, :, 1]

    srctab, cs = _pre_call(Xp, Wt, AW, bp, maskf)

    hmn0, hmn1, mrow = _edge_call(target_nodes, nb0, cur0, nb1, cur1,
                                  srctab, cs)

    hm0, hm1 = _gath_call(hmn0, hmn1, mrow)

    logits, emb = _post_call(hm0, hm1, W_sem, b_sem[None, :], a_sem,
                             W_cls, b_cls[None, :])
    return (logits, emb)
